# baseline (speedup 1.0000x reference)
import functools

import jax
import jax.numpy as jnp
from jax import lax
from jax.experimental import pallas as pl
from jax.experimental.pallas import tpu as pltpu
from jax.experimental.pallas import tpu_sc as plsc

NC = 2
NS = 16
NW = NC * NS
CHUNK = 128
NBUF = 3


def _mesh():
  return plsc.VectorSubcoreMesh(
      core_axis_name="c", subcore_axis_name="s", num_cores=NC,
      num_subcores=NS)


def _deg_call(src, dst, n_pad, e2):
  epw = e2 // NW
  n_chunks = epw // CHUNK
  n_groups = n_chunks // NBUF
  rows_per_tile = n_pad // NS
  zeros = jnp.zeros((rows_per_tile, 2), jnp.float32)
  ones_src = jnp.tile(jnp.array([[1.0, 0.0]], jnp.float32), (CHUNK, 1))
  ones_dst = jnp.tile(jnp.array([[0.0, 1.0]], jnp.float32), (CHUNK, 1))

  @functools.partial(
      pl.kernel,
      out_type=jax.ShapeDtypeStruct((NC, n_pad, 2), jnp.float32),
      mesh=_mesh(),
      scratch_types=[
          [pltpu.VMEM((CHUNK,), jnp.int32) for _ in range(NBUF)],
          [pltpu.VMEM((CHUNK,), jnp.int32) for _ in range(NBUF)],
          pltpu.VMEM((CHUNK, 2), jnp.float32),
          pltpu.VMEM((CHUNK, 2), jnp.float32),
          pltpu.VMEM_SHARED((n_pad, 2), jnp.float32),
          [pltpu.SemaphoreType.DMA for _ in range(NBUF)],
          [pltpu.SemaphoreType.DMA for _ in range(NBUF)],
      ],
  )
  def deg_kernel(src_hbm, dst_hbm, zz_hbm, os_hbm, od_hbm, out_hbm, idx_s,
                 idx_d, ones_s, ones_d, acc, isem, ssem):
    cid = lax.axis_index("c")
    sid = lax.axis_index("s")
    wid = sid * NC + cid
    base0 = wid * epw
    pltpu.sync_copy(zz_hbm, acc.at[pl.ds(sid * rows_per_tile,
                                         rows_per_tile)])
    pltpu.sync_copy(os_hbm, ones_s)
    pltpu.sync_copy(od_hbm, ones_d)
    plsc.subcore_barrier()

    def fire_idx(c, k):
      base = base0 + c * CHUNK
      pltpu.async_copy(src_hbm.at[pl.ds(base, CHUNK)], idx_s[k], isem[k])
      pltpu.async_copy(dst_hbm.at[pl.ds(base, CHUNK)], idx_d[k], isem[k])

    def wait_idx(k):
      pltpu.make_async_copy(src_hbm.at[pl.ds(0, CHUNK)], idx_s[k],
                            isem[k]).wait()
      pltpu.make_async_copy(dst_hbm.at[pl.ds(0, CHUNK)], idx_d[k],
                            isem[k]).wait()

    def group(g, refire):
      for k in range(NBUF):
        wait_idx(k)
        pltpu.async_copy(ones_s, acc.at[idx_s[k]], ssem[k], add=True)
        pltpu.async_copy(ones_d, acc.at[idx_d[k]], ssem[k], add=True)
      for k in range(NBUF):
        pltpu.make_async_copy(ones_s, acc.at[idx_s[k]], ssem[k]).wait()
        pltpu.make_async_copy(ones_d, acc.at[idx_d[k]], ssem[k]).wait()
        if refire:
          fire_idx((g + 1) * NBUF + k, k)

    for k in range(NBUF):
      fire_idx(k, k)
    lax.fori_loop(0, n_groups - 1, lambda g, _: (group(g, True), 0)[1], 0)
    group(n_groups - 1, False)

    plsc.subcore_barrier()
    sl = pl.ds(sid * rows_per_tile, rows_per_tile)
    pltpu.sync_copy(acc.at[sl], out_hbm.at[cid, sl, :])

  return deg_kernel(src, dst, zeros, ones_src, ones_dst)


def _scatter_call(table, src, dst, n2, d, e2):
  epw = e2 // NW
  n_chunks = epw // CHUNK
  n_groups = n_chunks // NBUF
  rows_per_tile = n2 // NS
  zeros = jnp.zeros((rows_per_tile, d), jnp.float32)

  @functools.partial(
      pl.kernel,
      out_type=jax.ShapeDtypeStruct((NC, n2, d), jnp.float32),
      mesh=_mesh(),
      scratch_types=[
          [pltpu.VMEM((CHUNK,), jnp.int32) for _ in range(NBUF)],
          [pltpu.VMEM((CHUNK,), jnp.int32) for _ in range(NBUF)],
          [pltpu.VMEM((CHUNK, d), jnp.float32) for _ in range(NBUF)],
          pltpu.VMEM_SHARED((n2, d), jnp.float32),
          [pltpu.SemaphoreType.DMA for _ in range(NBUF)],
          [pltpu.SemaphoreType.DMA for _ in range(NBUF)],
          [pltpu.SemaphoreType.DMA for _ in range(NBUF)],
      ],
  )
  def scat_kernel(table_hbm, src_hbm, dst_hbm, zz_hbm, out_hbm, idx_s, idx_d,
                  rows_v, acc, isem, gsem, ssem):
    cid = lax.axis_index("c")
    sid = lax.axis_index("s")
    wid = sid * NC + cid
    base0 = wid * epw
    pltpu.sync_copy(zz_hbm, acc.at[pl.ds(sid * rows_per_tile,
                                         rows_per_tile)])
    plsc.subcore_barrier()

    def fire(c, k):
      base = base0 + c * CHUNK
      pltpu.async_copy(src_hbm.at[pl.ds(base, CHUNK)], idx_s[k], isem[k])
      pltpu.async_copy(dst_hbm.at[pl.ds(base, CHUNK)], idx_d[k], isem[k])

    def group(g, refire):
      for k in range(NBUF):
        pltpu.make_async_copy(src_hbm.at[pl.ds(0, CHUNK)], idx_s[k],
                              isem[k]).wait()
        pltpu.make_async_copy(dst_hbm.at[pl.ds(0, CHUNK)], idx_d[k],
                              isem[k]).wait()
        pltpu.async_copy(table_hbm.at[idx_s[k]], rows_v[k], gsem[k])
      for k in range(NBUF):
        pltpu.make_async_copy(table_hbm.at[idx_s[k]], rows_v[k],
                              gsem[k]).wait()
        pltpu.async_copy(rows_v[k], acc.at[idx_d[k]], ssem[k], add=True)
      for k in range(NBUF):
        pltpu.make_async_copy(rows_v[k], acc.at[idx_d[k]], ssem[k]).wait()
        if refire:
          fire((g + 1) * NBUF + k, k)

    for k in range(NBUF):
      fire(k, k)
    lax.fori_loop(0, n_groups - 1, lambda g, _: (group(g, True), 0)[1], 0)
    group(n_groups - 1, False)

    plsc.subcore_barrier()
    sl = pl.ds(sid * rows_per_tile, rows_per_tile)
    pltpu.sync_copy(acc.at[sl], out_hbm.at[cid, sl])

  return scat_kernel(table, src, dst, zeros)


_BLK = 1000


def _h_scaled_kernel(x_ref, w1_ref, deg_ref, out_ref):
  norm = lax.rsqrt(jnp.maximum(deg_ref[...], 1.0))
  h = jnp.dot(x_ref[...], w1_ref[...], preferred_element_type=jnp.float32,
              precision=lax.Precision.HIGHEST)
  out_ref[...] = h * norm


def _h1_kernel(aggp_ref, deg_ref, b1_ref, out_ref):
  agg = aggp_ref[0] + aggp_ref[1]
  norm = lax.rsqrt(jnp.maximum(deg_ref[...], 1.0))
  out_ref[...] = jnp.maximum(agg * norm + b1_ref[...], 0.0)


def _out_kernel(h1_ref, nsp_ref, deg_ref, ws_ref, wn_ref, b2_ref, out_ref):
  inv = 1.0 / jnp.maximum(deg_ref[...], 1.0)
  neigh = (nsp_ref[0] + nsp_ref[1]) * inv
  out_ref[...] = (
      jnp.dot(h1_ref[...], ws_ref[...], preferred_element_type=jnp.float32,
              precision=lax.Precision.HIGHEST)
      + jnp.dot(neigh, wn_ref[...], preferred_element_type=jnp.float32,
                precision=lax.Precision.HIGHEST)
      + b2_ref[...])


def kernel(x, edge_index, W1, b1, W_self, W_neigh, b2):
  n, d_in = x.shape
  e = edge_index.shape[1]
  d_hid = W1.shape[1]
  d_out = W_self.shape[1]
  src = edge_index[0]
  dst = edge_index[1]

  n_pad = ((n + 8 * NS) // (8 * NS)) * (8 * NS)
  n2 = n_pad

  step = NW * CHUNK * NBUF
  e2 = ((e + step - 1) // step) * step
  pad = e2 - e
  pad_lo = jnp.arange(pad, dtype=jnp.int32) % n
  pad_hi = n + jnp.arange(pad, dtype=jnp.int32) % (n_pad - n)
  src_deg = jnp.concatenate([src, pad_hi])
  src_gat = jnp.concatenate([src, pad_lo])
  dst_p = jnp.concatenate([dst, pad_hi])

  deg_parts = _deg_call(src_deg, dst_p, n_pad, e2)
  deg_out_col = (deg_parts[0, :n, 0] + deg_parts[1, :n, 0])[:, None]
  deg_in_col = (deg_parts[0, :n, 1] + deg_parts[1, :n, 1])[:, None]

  grid = n // _BLK
  hs = pl.pallas_call(
      _h_scaled_kernel,
      grid=(grid,),
      in_specs=[
          pl.BlockSpec((_BLK, d_in), lambda i: (i, 0)),
          pl.BlockSpec((d_in, d_hid), lambda i: (0, 0)),
          pl.BlockSpec((_BLK, 1), lambda i: (i, 0)),
      ],
      out_specs=pl.BlockSpec((_BLK, d_hid), lambda i: (i, 0)),
      out_shape=jax.ShapeDtypeStruct((n, d_hid), jnp.float32),
  )(x, W1, deg_out_col)

  agg_parts = _scatter_call(hs, src_gat, dst_p, n2, d_hid, e2)

  h1 = pl.pallas_call(
      _h1_kernel,
      grid=(grid,),
      in_specs=[
          pl.BlockSpec((2, _BLK, d_hid), lambda i: (0, i, 0)),
          pl.BlockSpec((_BLK, 1), lambda i: (i, 0)),
          pl.BlockSpec((d_hid,), lambda i: (0,)),
      ],
      out_specs=pl.BlockSpec((_BLK, d_hid), lambda i: (i, 0)),
      out_shape=jax.ShapeDtypeStruct((n, d_hid), jnp.float32),
  )(agg_parts, deg_in_col, b1)

  ns_parts = _scatter_call(h1, src_gat, dst_p, n2, d_hid, e2)

  out = pl.pallas_call(
      _out_kernel,
      grid=(grid,),
      in_specs=[
          pl.BlockSpec((_BLK, d_hid), lambda i: (i, 0)),
          pl.BlockSpec((2, _BLK, d_hid), lambda i: (0, i, 0)),
          pl.BlockSpec((_BLK, 1), lambda i: (i, 0)),
          pl.BlockSpec((d_hid, d_out), lambda i: (0, 0)),
          pl.BlockSpec((d_hid, d_out), lambda i: (0, 0)),
          pl.BlockSpec((d_out,), lambda i: (0,)),
      ],
      out_specs=pl.BlockSpec((_BLK, d_out), lambda i: (i, 0)),
      out_shape=jax.ShapeDtypeStruct((n, d_out), jnp.float32),
  )(h1, ns_parts, deg_in_col, W_self, W_neigh, b2)

  return out


# speedup vs baseline: 1.1115x; 1.1115x over previous
import functools

import jax
import jax.numpy as jnp
from jax import lax
from jax.experimental import pallas as pl
from jax.experimental.pallas import tpu as pltpu
from jax.experimental.pallas import tpu_sc as plsc

NC = 2
NS = 16
NW = NC * NS
CHUNK = 128
NBUF = 3


def _mesh():
  return plsc.VectorSubcoreMesh(
      core_axis_name="c", subcore_axis_name="s", num_cores=NC,
      num_subcores=NS)


def _deg_call(src, dst, n_pad, e2):
  epw = e2 // NW
  n_chunks = epw // CHUNK
  n_groups = n_chunks // NBUF
  rows_per_tile = n_pad // NS
  zeros = jnp.zeros((rows_per_tile, 2), jnp.float32)
  ones_src = jnp.tile(jnp.array([[1.0, 0.0]], jnp.float32), (CHUNK, 1))
  ones_dst = jnp.tile(jnp.array([[0.0, 1.0]], jnp.float32), (CHUNK, 1))

  @functools.partial(
      pl.kernel,
      out_type=jax.ShapeDtypeStruct((NC, n_pad, 2), jnp.float32),
      mesh=_mesh(),
      scratch_types=[
          [pltpu.VMEM((CHUNK,), jnp.int32) for _ in range(NBUF)],
          [pltpu.VMEM((CHUNK,), jnp.int32) for _ in range(NBUF)],
          pltpu.VMEM((CHUNK, 2), jnp.float32),
          pltpu.VMEM((CHUNK, 2), jnp.float32),
          pltpu.VMEM_SHARED((n_pad, 2), jnp.float32),
          [pltpu.SemaphoreType.DMA for _ in range(NBUF)],
          [pltpu.SemaphoreType.DMA for _ in range(NBUF)],
      ],
  )
  def deg_kernel(src_hbm, dst_hbm, zz_hbm, os_hbm, od_hbm, out_hbm, idx_s,
                 idx_d, ones_s, ones_d, acc, isem, ssem):
    cid = lax.axis_index("c")
    sid = lax.axis_index("s")
    wid = sid * NC + cid
    base0 = wid * epw
    pltpu.sync_copy(zz_hbm, acc.at[pl.ds(sid * rows_per_tile,
                                         rows_per_tile)])
    pltpu.sync_copy(os_hbm, ones_s)
    pltpu.sync_copy(od_hbm, ones_d)
    plsc.subcore_barrier()

    def fire_idx(c, k):
      base = base0 + c * CHUNK
      pltpu.async_copy(src_hbm.at[pl.ds(base, CHUNK)], idx_s[k], isem[k])
      pltpu.async_copy(dst_hbm.at[pl.ds(base, CHUNK)], idx_d[k], isem[k])

    def wait_idx(k):
      pltpu.make_async_copy(src_hbm.at[pl.ds(0, CHUNK)], idx_s[k],
                            isem[k]).wait()
      pltpu.make_async_copy(dst_hbm.at[pl.ds(0, CHUNK)], idx_d[k],
                            isem[k]).wait()

    def group(g, refire):
      for k in range(NBUF):
        wait_idx(k)
        pltpu.async_copy(ones_s, acc.at[idx_s[k]], ssem[k], add=True)
        pltpu.async_copy(ones_d, acc.at[idx_d[k]], ssem[k], add=True)
      for k in range(NBUF):
        pltpu.make_async_copy(ones_s, acc.at[idx_s[k]], ssem[k]).wait()
        pltpu.make_async_copy(ones_d, acc.at[idx_d[k]], ssem[k]).wait()
        if refire:
          fire_idx((g + 1) * NBUF + k, k)

    for k in range(NBUF):
      fire_idx(k, k)
    lax.fori_loop(0, n_groups - 1, lambda g, _: (group(g, True), 0)[1], 0)
    group(n_groups - 1, False)

    plsc.subcore_barrier()
    sl = pl.ds(sid * rows_per_tile, rows_per_tile)
    pltpu.sync_copy(acc.at[sl], out_hbm.at[cid, sl, :])

  return deg_kernel(src, dst, zeros, ones_src, ones_dst)


def _scatter_call(table, src, dst, n2, d, e2):
  epw = e2 // NW
  n_chunks = epw // CHUNK
  n_groups = n_chunks // NBUF
  rows_per_tile = n2 // NS
  zeros = jnp.zeros((rows_per_tile, d), jnp.float32)

  @functools.partial(
      pl.kernel,
      out_type=jax.ShapeDtypeStruct((NC, n2, d), jnp.float32),
      mesh=_mesh(),
      scratch_types=[
          [pltpu.VMEM((CHUNK,), jnp.int32) for _ in range(NBUF)],
          [pltpu.VMEM((CHUNK,), jnp.int32) for _ in range(NBUF)],
          [pltpu.VMEM((CHUNK, d), jnp.float32) for _ in range(NBUF)],
          pltpu.VMEM_SHARED((n2, d), jnp.float32),
          [pltpu.SemaphoreType.DMA for _ in range(NBUF)],
          [pltpu.SemaphoreType.DMA for _ in range(NBUF)],
          [pltpu.SemaphoreType.DMA for _ in range(NBUF)],
      ],
  )
  def scat_kernel(table_hbm, src_hbm, dst_hbm, zz_hbm, out_hbm, idx_s, idx_d,
                  rows_v, acc, isem, gsem, ssem):
    cid = lax.axis_index("c")
    sid = lax.axis_index("s")
    wid = sid * NC + cid
    base0 = wid * epw
    pltpu.sync_copy(zz_hbm, acc.at[pl.ds(sid * rows_per_tile,
                                         rows_per_tile)])
    plsc.subcore_barrier()

    def fire_idx(c, k):
      base = base0 + c * CHUNK
      pltpu.async_copy(src_hbm.at[pl.ds(base, CHUNK)], idx_s[k], isem[k])
      pltpu.async_copy(dst_hbm.at[pl.ds(base, CHUNK)], idx_d[k], isem[k])

    def wait_idx(k):
      pltpu.make_async_copy(src_hbm.at[pl.ds(0, CHUNK)], idx_s[k],
                            isem[k]).wait()
      pltpu.make_async_copy(dst_hbm.at[pl.ds(0, CHUNK)], idx_d[k],
                            isem[k]).wait()

    def fire_gather(k):
      pltpu.async_copy(table_hbm.at[idx_s[k]], rows_v[k], gsem[k])

    def wait_gather(k):
      pltpu.make_async_copy(table_hbm.at[idx_s[k]], rows_v[k],
                            gsem[k]).wait()

    def fire_scatter(k):
      pltpu.async_copy(rows_v[k], acc.at[idx_d[k]], ssem[k], add=True)

    def wait_scatter(k):
      pltpu.make_async_copy(rows_v[k], acc.at[idx_d[k]], ssem[k]).wait()

    for k in range(NBUF):
      fire_idx(k, k)
    for k in range(NBUF):
      wait_idx(k)
      fire_gather(k)
    for k in range(NBUF):
      wait_gather(k)
      fire_scatter(k)
      fire_idx(NBUF + k, k)

    def body(g, _):
      for k in range(NBUF):
        wait_scatter(k)
        wait_idx(k)
        fire_gather(k)
      for k in range(NBUF):
        wait_gather(k)
        fire_scatter(k)

        @pl.when(g < n_groups - 2)
        def _():
          fire_idx((g + 2) * NBUF + k, k)

      return 0

    lax.fori_loop(0, n_groups - 1, body, 0)
    for k in range(NBUF):
      wait_scatter(k)

    plsc.subcore_barrier()
    sl = pl.ds(sid * rows_per_tile, rows_per_tile)
    pltpu.sync_copy(acc.at[sl], out_hbm.at[cid, sl])

  return scat_kernel(table, src, dst, zeros)


_BLK = 1000


def _h_scaled_kernel(x_ref, w1_ref, deg_ref, out_ref):
  norm = lax.rsqrt(jnp.maximum(deg_ref[...], 1.0))
  h = jnp.dot(x_ref[...], w1_ref[...], preferred_element_type=jnp.float32,
              precision=lax.Precision.HIGHEST)
  out_ref[...] = h * norm


def _h1_kernel(aggp_ref, deg_ref, b1_ref, out_ref):
  agg = aggp_ref[0] + aggp_ref[1]
  norm = lax.rsqrt(jnp.maximum(deg_ref[...], 1.0))
  out_ref[...] = jnp.maximum(agg * norm + b1_ref[...], 0.0)


def _out_kernel(h1_ref, nsp_ref, deg_ref, ws_ref, wn_ref, b2_ref, out_ref):
  inv = 1.0 / jnp.maximum(deg_ref[...], 1.0)
  neigh = (nsp_ref[0] + nsp_ref[1]) * inv
  out_ref[...] = (
      jnp.dot(h1_ref[...], ws_ref[...], preferred_element_type=jnp.float32,
              precision=lax.Precision.HIGHEST)
      + jnp.dot(neigh, wn_ref[...], preferred_element_type=jnp.float32,
                precision=lax.Precision.HIGHEST)
      + b2_ref[...])


def kernel(x, edge_index, W1, b1, W_self, W_neigh, b2):
  n, d_in = x.shape
  e = edge_index.shape[1]
  d_hid = W1.shape[1]
  d_out = W_self.shape[1]
  src = edge_index[0]
  dst = edge_index[1]

  n_pad = ((n + 8 * NS) // (8 * NS)) * (8 * NS)
  n2 = n_pad

  step = NW * CHUNK * NBUF
  e2 = ((e + step - 1) // step) * step
  pad = e2 - e
  pad_lo = jnp.arange(pad, dtype=jnp.int32) % n
  pad_hi = n + jnp.arange(pad, dtype=jnp.int32) % (n_pad - n)
  src_deg = jnp.concatenate([src, pad_hi])
  src_gat = jnp.concatenate([src, pad_lo])
  dst_p = jnp.concatenate([dst, pad_hi])

  deg_parts = _deg_call(src_deg, dst_p, n_pad, e2)
  deg_out_col = (deg_parts[0, :n, 0] + deg_parts[1, :n, 0])[:, None]
  deg_in_col = (deg_parts[0, :n, 1] + deg_parts[1, :n, 1])[:, None]

  grid = n // _BLK
  hs = pl.pallas_call(
      _h_scaled_kernel,
      grid=(grid,),
      in_specs=[
          pl.BlockSpec((_BLK, d_in), lambda i: (i, 0)),
          pl.BlockSpec((d_in, d_hid), lambda i: (0, 0)),
          pl.BlockSpec((_BLK, 1), lambda i: (i, 0)),
      ],
      out_specs=pl.BlockSpec((_BLK, d_hid), lambda i: (i, 0)),
      out_shape=jax.ShapeDtypeStruct((n, d_hid), jnp.float32),
  )(x, W1, deg_out_col)

  agg_parts = _scatter_call(hs, src_gat, dst_p, n2, d_hid, e2)

  h1 = pl.pallas_call(
      _h1_kernel,
      grid=(grid,),
      in_specs=[
          pl.BlockSpec((2, _BLK, d_hid), lambda i: (0, i, 0)),
          pl.BlockSpec((_BLK, 1), lambda i: (i, 0)),
          pl.BlockSpec((d_hid,), lambda i: (0,)),
      ],
      out_specs=pl.BlockSpec((_BLK, d_hid), lambda i: (i, 0)),
      out_shape=jax.ShapeDtypeStruct((n, d_hid), jnp.float32),
  )(agg_parts, deg_in_col, b1)

  ns_parts = _scatter_call(h1, src_gat, dst_p, n2, d_hid, e2)

  out = pl.pallas_call(
      _out_kernel,
      grid=(grid,),
      in_specs=[
          pl.BlockSpec((_BLK, d_hid), lambda i: (i, 0)),
          pl.BlockSpec((2, _BLK, d_hid), lambda i: (0, i, 0)),
          pl.BlockSpec((_BLK, 1), lambda i: (i, 0)),
          pl.BlockSpec((d_hid, d_out), lambda i: (0, 0)),
          pl.BlockSpec((d_hid, d_out), lambda i: (0, 0)),
          pl.BlockSpec((d_out,), lambda i: (0,)),
      ],
      out_specs=pl.BlockSpec((_BLK, d_out), lambda i: (i, 0)),
      out_shape=jax.ShapeDtypeStruct((n, d_out), jnp.float32),
  )(h1, ns_parts, deg_in_col, W_self, W_neigh, b2)

  return out


# speedup vs baseline: 1.1396x; 1.0253x over previous
import functools

import jax
import jax.numpy as jnp
from jax import lax
from jax.experimental import pallas as pl
from jax.experimental.pallas import tpu as pltpu
from jax.experimental.pallas import tpu_sc as plsc

NC = 2
NS = 16
NW = NC * NS
CHUNK = 64
NBUF = 4


def _mesh():
  return plsc.VectorSubcoreMesh(
      core_axis_name="c", subcore_axis_name="s", num_cores=NC,
      num_subcores=NS)


def _deg_call(src, dst, n_pad, e2):
  epw = e2 // NW
  n_chunks = epw // CHUNK
  n_groups = n_chunks // NBUF
  rows_per_tile = n_pad // NS
  zeros = jnp.zeros((rows_per_tile, 2), jnp.float32)
  ones_src = jnp.tile(jnp.array([[1.0, 0.0]], jnp.float32), (CHUNK, 1))
  ones_dst = jnp.tile(jnp.array([[0.0, 1.0]], jnp.float32), (CHUNK, 1))

  @functools.partial(
      pl.kernel,
      out_type=jax.ShapeDtypeStruct((NC, n_pad, 2), jnp.float32),
      mesh=_mesh(),
      scratch_types=[
          [pltpu.VMEM((CHUNK,), jnp.int32) for _ in range(NBUF)],
          [pltpu.VMEM((CHUNK,), jnp.int32) for _ in range(NBUF)],
          pltpu.VMEM((CHUNK, 2), jnp.float32),
          pltpu.VMEM((CHUNK, 2), jnp.float32),
          pltpu.VMEM_SHARED((n_pad, 2), jnp.float32),
          [pltpu.SemaphoreType.DMA for _ in range(NBUF)],
          [pltpu.SemaphoreType.DMA for _ in range(NBUF)],
      ],
  )
  def deg_kernel(src_hbm, dst_hbm, zz_hbm, os_hbm, od_hbm, out_hbm, idx_s,
                 idx_d, ones_s, ones_d, acc, isem, ssem):
    cid = lax.axis_index("c")
    sid = lax.axis_index("s")
    wid = sid * NC + cid
    base0 = wid * epw
    pltpu.sync_copy(zz_hbm, acc.at[pl.ds(sid * rows_per_tile,
                                         rows_per_tile)])
    pltpu.sync_copy(os_hbm, ones_s)
    pltpu.sync_copy(od_hbm, ones_d)
    plsc.subcore_barrier()

    def fire_idx(c, k):
      base = base0 + c * CHUNK
      pltpu.async_copy(src_hbm.at[pl.ds(base, CHUNK)], idx_s[k], isem[k])
      pltpu.async_copy(dst_hbm.at[pl.ds(base, CHUNK)], idx_d[k], isem[k])

    def wait_idx(k):
      pltpu.make_async_copy(src_hbm.at[pl.ds(0, CHUNK)], idx_s[k],
                            isem[k]).wait()
      pltpu.make_async_copy(dst_hbm.at[pl.ds(0, CHUNK)], idx_d[k],
                            isem[k]).wait()

    def group(g, refire):
      for k in range(NBUF):
        wait_idx(k)
        pltpu.async_copy(ones_s, acc.at[idx_s[k]], ssem[k], add=True)
        pltpu.async_copy(ones_d, acc.at[idx_d[k]], ssem[k], add=True)
      for k in range(NBUF):
        pltpu.make_async_copy(ones_s, acc.at[idx_s[k]], ssem[k]).wait()
        pltpu.make_async_copy(ones_d, acc.at[idx_d[k]], ssem[k]).wait()
        if refire:
          fire_idx((g + 1) * NBUF + k, k)

    for k in range(NBUF):
      fire_idx(k, k)
    lax.fori_loop(0, n_groups - 1, lambda g, _: (group(g, True), 0)[1], 0)
    group(n_groups - 1, False)

    plsc.subcore_barrier()
    sl = pl.ds(sid * rows_per_tile, rows_per_tile)
    pltpu.sync_copy(acc.at[sl], out_hbm.at[cid, sl, :])

  return deg_kernel(src, dst, zeros, ones_src, ones_dst)


def _scatter_call(table, src, dst, n2, d, e2):
  epw = e2 // NW
  n_chunks = epw // CHUNK
  n_groups = n_chunks // NBUF
  rows_per_tile = n2 // NS
  zeros = jnp.zeros((rows_per_tile, d), jnp.float32)

  @functools.partial(
      pl.kernel,
      out_type=jax.ShapeDtypeStruct((NC, n2, d), jnp.float32),
      mesh=_mesh(),
      scratch_types=[
          [pltpu.VMEM((CHUNK,), jnp.int32) for _ in range(NBUF)],
          [pltpu.VMEM((CHUNK,), jnp.int32) for _ in range(NBUF)],
          [pltpu.VMEM((CHUNK, d), jnp.float32) for _ in range(NBUF)],
          pltpu.VMEM_SHARED((n2, d), jnp.float32),
          [pltpu.SemaphoreType.DMA for _ in range(NBUF)],
          [pltpu.SemaphoreType.DMA for _ in range(NBUF)],
          [pltpu.SemaphoreType.DMA for _ in range(NBUF)],
      ],
  )
  def scat_kernel(table_hbm, src_hbm, dst_hbm, zz_hbm, out_hbm, idx_s, idx_d,
                  rows_v, acc, isem, gsem, ssem):
    cid = lax.axis_index("c")
    sid = lax.axis_index("s")
    wid = sid * NC + cid
    base0 = wid * epw
    pltpu.sync_copy(zz_hbm, acc.at[pl.ds(sid * rows_per_tile,
                                         rows_per_tile)])
    plsc.subcore_barrier()

    def fire_idx(c, k):
      base = base0 + c * CHUNK
      pltpu.async_copy(src_hbm.at[pl.ds(base, CHUNK)], idx_s[k], isem[k])
      pltpu.async_copy(dst_hbm.at[pl.ds(base, CHUNK)], idx_d[k], isem[k])

    def wait_idx(k):
      pltpu.make_async_copy(src_hbm.at[pl.ds(0, CHUNK)], idx_s[k],
                            isem[k]).wait()
      pltpu.make_async_copy(dst_hbm.at[pl.ds(0, CHUNK)], idx_d[k],
                            isem[k]).wait()

    def fire_gather(k):
      pltpu.async_copy(table_hbm.at[idx_s[k]], rows_v[k], gsem[k])

    def wait_gather(k):
      pltpu.make_async_copy(table_hbm.at[idx_s[k]], rows_v[k],
                            gsem[k]).wait()

    def fire_scatter(k):
      pltpu.async_copy(rows_v[k], acc.at[idx_d[k]], ssem[k], add=True)

    def wait_scatter(k):
      pltpu.make_async_copy(rows_v[k], acc.at[idx_d[k]], ssem[k]).wait()

    for k in range(NBUF):
      fire_idx(k, k)
    for k in range(NBUF):
      wait_idx(k)
      fire_gather(k)
    for k in range(NBUF):
      wait_gather(k)
      fire_scatter(k)
      fire_idx(NBUF + k, k)

    def body(g, _):
      for k in range(NBUF):
        wait_scatter(k)
        wait_idx(k)
        fire_gather(k)
      for k in range(NBUF):
        wait_gather(k)
        fire_scatter(k)

        @pl.when(g < n_groups - 2)
        def _():
          fire_idx((g + 2) * NBUF + k, k)

      return 0

    lax.fori_loop(0, n_groups - 1, body, 0)
    for k in range(NBUF):
      wait_scatter(k)

    plsc.subcore_barrier()
    sl = pl.ds(sid * rows_per_tile, rows_per_tile)
    pltpu.sync_copy(acc.at[sl], out_hbm.at[cid, sl])

  return scat_kernel(table, src, dst, zeros)


_BLK = 1000


def _h_scaled_kernel(x_ref, w1_ref, deg_ref, out_ref):
  norm = lax.rsqrt(jnp.maximum(deg_ref[...], 1.0))
  h = jnp.dot(x_ref[...], w1_ref[...], preferred_element_type=jnp.float32,
              precision=lax.Precision.HIGHEST)
  out_ref[...] = h * norm


def _h1_kernel(aggp_ref, deg_ref, b1_ref, out_ref):
  agg = aggp_ref[0] + aggp_ref[1]
  norm = lax.rsqrt(jnp.maximum(deg_ref[...], 1.0))
  out_ref[...] = jnp.maximum(agg * norm + b1_ref[...], 0.0)


def _out_kernel(h1_ref, nsp_ref, deg_ref, ws_ref, wn_ref, b2_ref, out_ref):
  inv = 1.0 / jnp.maximum(deg_ref[...], 1.0)
  neigh = (nsp_ref[0] + nsp_ref[1]) * inv
  out_ref[...] = (
      jnp.dot(h1_ref[...], ws_ref[...], preferred_element_type=jnp.float32,
              precision=lax.Precision.HIGHEST)
      + jnp.dot(neigh, wn_ref[...], preferred_element_type=jnp.float32,
                precision=lax.Precision.HIGHEST)
      + b2_ref[...])


def kernel(x, edge_index, W1, b1, W_self, W_neigh, b2):
  n, d_in = x.shape
  e = edge_index.shape[1]
  d_hid = W1.shape[1]
  d_out = W_self.shape[1]
  src = edge_index[0]
  dst = edge_index[1]

  n_pad = ((n + 8 * NS) // (8 * NS)) * (8 * NS)
  n2 = n_pad

  step = NW * CHUNK * NBUF
  e2 = ((e + step - 1) // step) * step
  pad = e2 - e
  pad_lo = jnp.arange(pad, dtype=jnp.int32) % n
  pad_hi = n + jnp.arange(pad, dtype=jnp.int32) % (n_pad - n)
  src_deg = jnp.concatenate([src, pad_hi])
  src_gat = jnp.concatenate([src, pad_lo])
  dst_p = jnp.concatenate([dst, pad_hi])

  deg_parts = _deg_call(src_deg, dst_p, n_pad, e2)
  deg_out_col = (deg_parts[0, :n, 0] + deg_parts[1, :n, 0])[:, None]
  deg_in_col = (deg_parts[0, :n, 1] + deg_parts[1, :n, 1])[:, None]

  grid = n // _BLK
  hs = pl.pallas_call(
      _h_scaled_kernel,
      grid=(grid,),
      in_specs=[
          pl.BlockSpec((_BLK, d_in), lambda i: (i, 0)),
          pl.BlockSpec((d_in, d_hid), lambda i: (0, 0)),
          pl.BlockSpec((_BLK, 1), lambda i: (i, 0)),
      ],
      out_specs=pl.BlockSpec((_BLK, d_hid), lambda i: (i, 0)),
      out_shape=jax.ShapeDtypeStruct((n, d_hid), jnp.float32),
  )(x, W1, deg_out_col)

  agg_parts = _scatter_call(hs, src_gat, dst_p, n2, d_hid, e2)

  h1 = pl.pallas_call(
      _h1_kernel,
      grid=(grid,),
      in_specs=[
          pl.BlockSpec((2, _BLK, d_hid), lambda i: (0, i, 0)),
          pl.BlockSpec((_BLK, 1), lambda i: (i, 0)),
          pl.BlockSpec((d_hid,), lambda i: (0,)),
      ],
      out_specs=pl.BlockSpec((_BLK, d_hid), lambda i: (i, 0)),
      out_shape=jax.ShapeDtypeStruct((n, d_hid), jnp.float32),
  )(agg_parts, deg_in_col, b1)

  ns_parts = _scatter_call(h1, src_gat, dst_p, n2, d_hid, e2)

  out = pl.pallas_call(
      _out_kernel,
      grid=(grid,),
      in_specs=[
          pl.BlockSpec((_BLK, d_hid), lambda i: (i, 0)),
          pl.BlockSpec((2, _BLK, d_hid), lambda i: (0, i, 0)),
          pl.BlockSpec((_BLK, 1), lambda i: (i, 0)),
          pl.BlockSpec((d_hid, d_out), lambda i: (0, 0)),
          pl.BlockSpec((d_hid, d_out), lambda i: (0, 0)),
          pl.BlockSpec((d_out,), lambda i: (0,)),
      ],
      out_specs=pl.BlockSpec((_BLK, d_out), lambda i: (i, 0)),
      out_shape=jax.ShapeDtypeStruct((n, d_out), jnp.float32),
  )(h1, ns_parts, deg_in_col, W_self, W_neigh, b2)

  return out


# speedup vs baseline: 1.2333x; 1.0823x over previous
import functools

import jax
import jax.numpy as jnp
from jax import lax
from jax.experimental import pallas as pl
from jax.experimental.pallas import tpu as pltpu
from jax.experimental.pallas import tpu_sc as plsc

NC = 2
NS = 16
NW = NC * NS
CHUNK = 64
NBUF = 4
NSLOT = 2 * NBUF


def _mesh():
  return plsc.VectorSubcoreMesh(
      core_axis_name="c", subcore_axis_name="s", num_cores=NC,
      num_subcores=NS)


def _deg_call(src, dst, n_pad, e2):
  epw = e2 // NW
  n_chunks = epw // CHUNK
  rows_per_tile = n_pad // NS
  zeros = jnp.zeros((rows_per_tile, 2), jnp.float32)
  ones_src = jnp.tile(jnp.array([[1.0, 0.0]], jnp.float32), (CHUNK, 1))
  ones_dst = jnp.tile(jnp.array([[0.0, 1.0]], jnp.float32), (CHUNK, 1))

  @functools.partial(
      pl.kernel,
      out_type=jax.ShapeDtypeStruct((NC, n_pad, 2), jnp.float32),
      mesh=_mesh(),
      scratch_types=[
          [pltpu.VMEM((CHUNK,), jnp.int32) for _ in range(NSLOT)],
          [pltpu.VMEM((CHUNK,), jnp.int32) for _ in range(NSLOT)],
          pltpu.VMEM((CHUNK, 2), jnp.float32),
          pltpu.VMEM((CHUNK, 2), jnp.float32),
          pltpu.VMEM_SHARED((n_pad, 2), jnp.float32),
          [pltpu.SemaphoreType.DMA for _ in range(NSLOT)],
          [pltpu.SemaphoreType.DMA for _ in range(NBUF)],
      ],
  )
  def deg_kernel(src_hbm, dst_hbm, zz_hbm, os_hbm, od_hbm, out_hbm, idx_s,
                 idx_d, ones_s, ones_d, acc, isem, ssem):
    cid = lax.axis_index("c")
    sid = lax.axis_index("s")
    wid = sid * NC + cid
    base0 = wid * epw
    pltpu.sync_copy(zz_hbm, acc.at[pl.ds(sid * rows_per_tile,
                                         rows_per_tile)])
    pltpu.sync_copy(os_hbm, ones_s)
    pltpu.sync_copy(od_hbm, ones_d)
    plsc.subcore_barrier()

    def fire_idx(c, k):
      base = base0 + c * CHUNK
      pltpu.async_copy(src_hbm.at[pl.ds(base, CHUNK)], idx_s[k], isem[k])
      pltpu.async_copy(dst_hbm.at[pl.ds(base, CHUNK)], idx_d[k], isem[k])

    def wait_idx(k):
      pltpu.make_async_copy(src_hbm.at[pl.ds(0, CHUNK)], idx_s[k],
                            isem[k]).wait()
      pltpu.make_async_copy(dst_hbm.at[pl.ds(0, CHUNK)], idx_d[k],
                            isem[k]).wait()

    def fire_scatter(k8, k4):
      pltpu.async_copy(ones_s, acc.at[idx_s[k8]], ssem[k4], add=True)
      pltpu.async_copy(ones_d, acc.at[idx_d[k8]], ssem[k4], add=True)

    def wait_scatter(k8, k4):
      pltpu.make_async_copy(ones_s, acc.at[idx_s[k8]], ssem[k4]).wait()
      pltpu.make_async_copy(ones_d, acc.at[idx_d[k8]], ssem[k4]).wait()

    def emit(cpos, k8, do_ws, do_i, do_s):
      if do_ws:
        wait_scatter((k8 - 2) % NSLOT, (k8 - 2) % NBUF)
      if do_i:
        fire_idx(cpos + 4, (k8 + 4) % NSLOT)
      if do_s:
        wait_idx(k8)
        fire_scatter(k8, k8 % NBUF)

    for c in range(4):
      fire_idx(c, c % NSLOT)
    for c in range(8):
      emit(c, c % NSLOT, c >= 2, c + 4 <= n_chunks - 1, True)

    def body(j, _):
      cpos = 8 + j * 8
      for k in range(8):
        emit(cpos + k, k, True, True, True)
      return 0

    lax.fori_loop(0, (n_chunks - 16) // 8, body, 0)
    for c in range(n_chunks - 8, n_chunks + 2):
      emit(c, c % NSLOT, True, c + 4 <= n_chunks - 1, c <= n_chunks - 1)

    plsc.subcore_barrier()
    sl = pl.ds(sid * rows_per_tile, rows_per_tile)
    pltpu.sync_copy(acc.at[sl], out_hbm.at[cid, sl, :])

  return deg_kernel(src, dst, zeros, ones_src, ones_dst)


def _scatter_call(table, src, dst, n2, d, e2):
  epw = e2 // NW
  n_chunks = epw // CHUNK
  rows_per_tile = n2 // NS
  zeros = jnp.zeros((rows_per_tile, d), jnp.float32)

  @functools.partial(
      pl.kernel,
      out_type=jax.ShapeDtypeStruct((NC, n2, d), jnp.float32),
      mesh=_mesh(),
      scratch_types=[
          [pltpu.VMEM((CHUNK,), jnp.int32) for _ in range(NSLOT)],
          [pltpu.VMEM((CHUNK,), jnp.int32) for _ in range(NSLOT)],
          [pltpu.VMEM((CHUNK, d), jnp.float32) for _ in range(NBUF)],
          pltpu.VMEM_SHARED((n2, d), jnp.float32),
          [pltpu.SemaphoreType.DMA for _ in range(NSLOT)],
          [pltpu.SemaphoreType.DMA for _ in range(NBUF)],
          [pltpu.SemaphoreType.DMA for _ in range(NBUF)],
      ],
  )
  def scat_kernel(table_hbm, src_hbm, dst_hbm, zz_hbm, out_hbm, idx_s, idx_d,
                  rows_v, acc, isem, gsem, ssem):
    cid = lax.axis_index("c")
    sid = lax.axis_index("s")
    wid = sid * NC + cid
    base0 = wid * epw
    pltpu.sync_copy(zz_hbm, acc.at[pl.ds(sid * rows_per_tile,
                                         rows_per_tile)])
    plsc.subcore_barrier()

    def fire_idx(c, k):
      base = base0 + c * CHUNK
      pltpu.async_copy(src_hbm.at[pl.ds(base, CHUNK)], idx_s[k], isem[k])
      pltpu.async_copy(dst_hbm.at[pl.ds(base, CHUNK)], idx_d[k], isem[k])

    def wait_idx(k):
      pltpu.make_async_copy(src_hbm.at[pl.ds(0, CHUNK)], idx_s[k],
                            isem[k]).wait()
      pltpu.make_async_copy(dst_hbm.at[pl.ds(0, CHUNK)], idx_d[k],
                            isem[k]).wait()

    def emit(cpos, k8, do_ws, do_i, do_g, do_s):
      k4 = k8 % NBUF
      if do_ws:
        pltpu.make_async_copy(rows_v[k4],
                              acc.at[idx_d[(k8 + 4) % NSLOT]],
                              ssem[k4]).wait()
      if do_i:
        fire_idx(cpos + 4, (k8 + 4) % NSLOT)
      if do_g:
        wait_idx(k8)
        pltpu.async_copy(table_hbm.at[idx_s[k8]], rows_v[k4], gsem[k4])
      if do_s:
        b = (k4 + 2) % NBUF
        s = (k8 + 6) % NSLOT
        pltpu.make_async_copy(table_hbm.at[idx_s[s]], rows_v[b],
                              gsem[b]).wait()
        pltpu.async_copy(rows_v[b], acc.at[idx_d[s]], ssem[b], add=True)

    for c in range(4):
      fire_idx(c, c % NSLOT)
    for c in range(8):
      emit(c, c % NSLOT, c >= 4, c + 4 <= n_chunks - 1, True, c >= 2)

    def body(j, _):
      cpos = 8 + j * 8
      for k in range(8):
        emit(cpos + k, k, True, True, True, True)
      return 0

    lax.fori_loop(0, (n_chunks - 16) // 8, body, 0)
    for c in range(n_chunks - 8, n_chunks + 4):
      emit(c, c % NSLOT, True, c + 4 <= n_chunks - 1, c <= n_chunks - 1,
           c <= n_chunks + 1)

    plsc.subcore_barrier()
    sl = pl.ds(sid * rows_per_tile, rows_per_tile)
    pltpu.sync_copy(acc.at[sl], out_hbm.at[cid, sl])

  return scat_kernel(table, src, dst, zeros)


_BLK = 1000


def _h_scaled_kernel(x_ref, w1_ref, deg_ref, out_ref):
  norm = lax.rsqrt(jnp.maximum(deg_ref[...], 1.0))
  h = jnp.dot(x_ref[...], w1_ref[...], preferred_element_type=jnp.float32,
              precision=lax.Precision.HIGHEST)
  out_ref[...] = h * norm


def _h1_kernel(aggp_ref, deg_ref, b1_ref, out_ref):
  agg = aggp_ref[0] + aggp_ref[1]
  norm = lax.rsqrt(jnp.maximum(deg_ref[...], 1.0))
  out_ref[...] = jnp.maximum(agg * norm + b1_ref[...], 0.0)


def _out_kernel(h1_ref, nsp_ref, deg_ref, ws_ref, wn_ref, b2_ref, out_ref):
  inv = 1.0 / jnp.maximum(deg_ref[...], 1.0)
  neigh = (nsp_ref[0] + nsp_ref[1]) * inv
  out_ref[...] = (
      jnp.dot(h1_ref[...], ws_ref[...], preferred_element_type=jnp.float32,
              precision=lax.Precision.HIGHEST)
      + jnp.dot(neigh, wn_ref[...], preferred_element_type=jnp.float32,
                precision=lax.Precision.HIGHEST)
      + b2_ref[...])


def kernel(x, edge_index, W1, b1, W_self, W_neigh, b2):
  n, d_in = x.shape
  e = edge_index.shape[1]
  d_hid = W1.shape[1]
  d_out = W_self.shape[1]
  src = edge_index[0]
  dst = edge_index[1]

  n_pad = ((n + 8 * NS) // (8 * NS)) * (8 * NS)
  n2 = n_pad

  step = NW * CHUNK * 8
  e2 = ((e + step - 1) // step) * step
  pad = e2 - e
  pad_lo = jnp.arange(pad, dtype=jnp.int32) % n
  pad_hi = n + jnp.arange(pad, dtype=jnp.int32) % (n_pad - n)
  src_deg = jnp.concatenate([src, pad_hi])
  src_gat = jnp.concatenate([src, pad_lo])
  dst_p = jnp.concatenate([dst, pad_hi])

  deg_parts = _deg_call(src_deg, dst_p, n_pad, e2)
  deg_out_col = (deg_parts[0, :n, 0] + deg_parts[1, :n, 0])[:, None]
  deg_in_col = (deg_parts[0, :n, 1] + deg_parts[1, :n, 1])[:, None]

  grid = n // _BLK
  hs = pl.pallas_call(
      _h_scaled_kernel,
      grid=(grid,),
      in_specs=[
          pl.BlockSpec((_BLK, d_in), lambda i: (i, 0)),
          pl.BlockSpec((d_in, d_hid), lambda i: (0, 0)),
          pl.BlockSpec((_BLK, 1), lambda i: (i, 0)),
      ],
      out_specs=pl.BlockSpec((_BLK, d_hid), lambda i: (i, 0)),
      out_shape=jax.ShapeDtypeStruct((n, d_hid), jnp.float32),
  )(x, W1, deg_out_col)

  agg_parts = _scatter_call(hs, src_gat, dst_p, n2, d_hid, e2)

  h1 = pl.pallas_call(
      _h1_kernel,
      grid=(grid,),
      in_specs=[
          pl.BlockSpec((2, _BLK, d_hid), lambda i: (0, i, 0)),
          pl.BlockSpec((_BLK, 1), lambda i: (i, 0)),
          pl.BlockSpec((d_hid,), lambda i: (0,)),
      ],
      out_specs=pl.BlockSpec((_BLK, d_hid), lambda i: (i, 0)),
      out_shape=jax.ShapeDtypeStruct((n, d_hid), jnp.float32),
  )(agg_parts, deg_in_col, b1)

  ns_parts = _scatter_call(h1, src_gat, dst_p, n2, d_hid, e2)

  out = pl.pallas_call(
      _out_kernel,
      grid=(grid,),
      in_specs=[
          pl.BlockSpec((_BLK, d_hid), lambda i: (i, 0)),
          pl.BlockSpec((2, _BLK, d_hid), lambda i: (0, i, 0)),
          pl.BlockSpec((_BLK, 1), lambda i: (i, 0)),
          pl.BlockSpec((d_hid, d_out), lambda i: (0, 0)),
          pl.BlockSpec((d_hid, d_out), lambda i: (0, 0)),
          pl.BlockSpec((d_out,), lambda i: (0,)),
      ],
      out_specs=pl.BlockSpec((_BLK, d_out), lambda i: (i, 0)),
      out_shape=jax.ShapeDtypeStruct((n, d_out), jnp.float32),
  )(h1, ns_parts, deg_in_col, W_self, W_neigh, b2)

  return out


# speedup vs baseline: 1.2646x; 1.0253x over previous
import functools

import jax
import jax.numpy as jnp
from jax import lax
from jax.experimental import pallas as pl
from jax.experimental.pallas import tpu as pltpu
from jax.experimental.pallas import tpu_sc as plsc

NC = 2
NS = 16
NW = NC * NS
CHUNK = 80
DEG_CHUNK = 128
NBUF = 4
NSLOT = 2 * NBUF


def _mesh():
  return plsc.VectorSubcoreMesh(
      core_axis_name="c", subcore_axis_name="s", num_cores=NC,
      num_subcores=NS)


def _deg_call(src, dst, n_pad, e2):
  epw = e2 // NW
  n_chunks = epw // DEG_CHUNK
  rows_per_tile = n_pad // NS
  zeros = jnp.zeros((rows_per_tile, 2), jnp.float32)
  ones_src = jnp.tile(jnp.array([[1.0, 0.0]], jnp.float32), (DEG_CHUNK, 1))
  ones_dst = jnp.tile(jnp.array([[0.0, 1.0]], jnp.float32), (DEG_CHUNK, 1))

  @functools.partial(
      pl.kernel,
      out_type=jax.ShapeDtypeStruct((NC, n_pad, 2), jnp.float32),
      mesh=_mesh(),
      scratch_types=[
          [pltpu.VMEM((DEG_CHUNK,), jnp.int32) for _ in range(NSLOT)],
          [pltpu.VMEM((DEG_CHUNK,), jnp.int32) for _ in range(NSLOT)],
          pltpu.VMEM((DEG_CHUNK, 2), jnp.float32),
          pltpu.VMEM((DEG_CHUNK, 2), jnp.float32),
          pltpu.VMEM_SHARED((n_pad, 2), jnp.float32),
          [pltpu.SemaphoreType.DMA for _ in range(NSLOT)],
          [pltpu.SemaphoreType.DMA for _ in range(NBUF)],
      ],
  )
  def deg_kernel(src_hbm, dst_hbm, zz_hbm, os_hbm, od_hbm, out_hbm, idx_s,
                 idx_d, ones_s, ones_d, acc, isem, ssem):
    cid = lax.axis_index("c")
    sid = lax.axis_index("s")
    wid = sid * NC + cid
    base0 = wid * epw
    pltpu.sync_copy(zz_hbm, acc.at[pl.ds(sid * rows_per_tile,
                                         rows_per_tile)])
    pltpu.sync_copy(os_hbm, ones_s)
    pltpu.sync_copy(od_hbm, ones_d)
    plsc.subcore_barrier()

    def fire_idx(c, k):
      base = base0 + c * DEG_CHUNK
      pltpu.async_copy(src_hbm.at[pl.ds(base, DEG_CHUNK)], idx_s[k], isem[k])
      pltpu.async_copy(dst_hbm.at[pl.ds(base, DEG_CHUNK)], idx_d[k], isem[k])

    def wait_idx(k):
      pltpu.make_async_copy(src_hbm.at[pl.ds(0, DEG_CHUNK)], idx_s[k],
                            isem[k]).wait()
      pltpu.make_async_copy(dst_hbm.at[pl.ds(0, DEG_CHUNK)], idx_d[k],
                            isem[k]).wait()

    def fire_scatter(k8, k4):
      pltpu.async_copy(ones_s, acc.at[idx_s[k8]], ssem[k4], add=True)
      pltpu.async_copy(ones_d, acc.at[idx_d[k8]], ssem[k4], add=True)

    def wait_scatter(k8, k4):
      pltpu.make_async_copy(ones_s, acc.at[idx_s[k8]], ssem[k4]).wait()
      pltpu.make_async_copy(ones_d, acc.at[idx_d[k8]], ssem[k4]).wait()

    def emit(cpos, k8, do_ws, do_i, do_s):
      if do_ws:
        wait_scatter((k8 - 2) % NSLOT, (k8 - 2) % NBUF)
      if do_i:
        fire_idx(cpos + 4, (k8 + 4) % NSLOT)
      if do_s:
        wait_idx(k8)
        fire_scatter(k8, k8 % NBUF)

    for c in range(4):
      fire_idx(c, c % NSLOT)
    for c in range(8):
      emit(c, c % NSLOT, c >= 2, c + 4 <= n_chunks - 1, True)

    def body(j, _):
      cpos = 8 + j * 8
      for k in range(8):
        emit(cpos + k, k, True, True, True)
      return 0

    lax.fori_loop(0, (n_chunks - 16) // 8, body, 0)
    for c in range(n_chunks - 8, n_chunks + 2):
      emit(c, c % NSLOT, True, c + 4 <= n_chunks - 1, c <= n_chunks - 1)

    plsc.subcore_barrier()
    sl = pl.ds(sid * rows_per_tile, rows_per_tile)
    pltpu.sync_copy(acc.at[sl], out_hbm.at[cid, sl, :])

  return deg_kernel(src, dst, zeros, ones_src, ones_dst)


def _scatter_call(table, src, dst, n2, d, e2):
  epw = e2 // NW
  n_chunks = epw // CHUNK
  rows_per_tile = n2 // NS
  zeros = jnp.zeros((rows_per_tile, d), jnp.float32)

  @functools.partial(
      pl.kernel,
      out_type=jax.ShapeDtypeStruct((NC, n2, d), jnp.float32),
      mesh=_mesh(),
      scratch_types=[
          [pltpu.VMEM((CHUNK,), jnp.int32) for _ in range(NSLOT)],
          [pltpu.VMEM((CHUNK,), jnp.int32) for _ in range(NSLOT)],
          [pltpu.VMEM((CHUNK, d), jnp.float32) for _ in range(NBUF)],
          pltpu.VMEM_SHARED((n2, d), jnp.float32),
          [pltpu.SemaphoreType.DMA for _ in range(NSLOT)],
          [pltpu.SemaphoreType.DMA for _ in range(NBUF)],
          [pltpu.SemaphoreType.DMA for _ in range(NBUF)],
      ],
  )
  def scat_kernel(table_hbm, src_hbm, dst_hbm, zz_hbm, out_hbm, idx_s, idx_d,
                  rows_v, acc, isem, gsem, ssem):
    cid = lax.axis_index("c")
    sid = lax.axis_index("s")
    wid = sid * NC + cid
    base0 = wid * epw
    pltpu.sync_copy(zz_hbm, acc.at[pl.ds(sid * rows_per_tile,
                                         rows_per_tile)])
    plsc.subcore_barrier()

    def fire_idx(c, k):
      base = base0 + c * CHUNK
      pltpu.async_copy(src_hbm.at[pl.ds(base, CHUNK)], idx_s[k], isem[k])
      pltpu.async_copy(dst_hbm.at[pl.ds(base, CHUNK)], idx_d[k], isem[k])

    def wait_idx(k):
      pltpu.make_async_copy(src_hbm.at[pl.ds(0, CHUNK)], idx_s[k],
                            isem[k]).wait()
      pltpu.make_async_copy(dst_hbm.at[pl.ds(0, CHUNK)], idx_d[k],
                            isem[k]).wait()

    def emit(cpos, k8, do_ws, do_i, do_g, do_s):
      k4 = k8 % NBUF
      if do_ws:
        pltpu.make_async_copy(rows_v[k4],
                              acc.at[idx_d[(k8 + 4) % NSLOT]],
                              ssem[k4]).wait()
      if do_i:
        fire_idx(cpos + 4, (k8 + 4) % NSLOT)
      if do_g:
        wait_idx(k8)
        pltpu.async_copy(table_hbm.at[idx_s[k8]], rows_v[k4], gsem[k4])
      if do_s:
        b = (k4 + 2) % NBUF
        s = (k8 + 6) % NSLOT
        pltpu.make_async_copy(table_hbm.at[idx_s[s]], rows_v[b],
                              gsem[b]).wait()
        pltpu.async_copy(rows_v[b], acc.at[idx_d[s]], ssem[b], add=True)

    for c in range(4):
      fire_idx(c, c % NSLOT)
    for c in range(8):
      emit(c, c % NSLOT, c >= 4, c + 4 <= n_chunks - 1, True, c >= 2)

    def body(j, _):
      cpos = 8 + j * 8
      for k in range(8):
        emit(cpos + k, k, True, True, True, True)
      return 0

    lax.fori_loop(0, (n_chunks - 16) // 8, body, 0)
    for c in range(n_chunks - 8, n_chunks + 4):
      emit(c, c % NSLOT, True, c + 4 <= n_chunks - 1, c <= n_chunks - 1,
           c <= n_chunks + 1)

    plsc.subcore_barrier()
    sl = pl.ds(sid * rows_per_tile, rows_per_tile)
    pltpu.sync_copy(acc.at[sl], out_hbm.at[cid, sl])

  return scat_kernel(table, src, dst, zeros)


_BLK = 1000


def _h_scaled_kernel(x_ref, w1_ref, deg_ref, out_ref):
  norm = lax.rsqrt(jnp.maximum(deg_ref[...], 1.0))
  h = jnp.dot(x_ref[...], w1_ref[...], preferred_element_type=jnp.float32,
              precision=lax.Precision.HIGHEST)
  out_ref[...] = h * norm


def _h1_kernel(aggp_ref, deg_ref, b1_ref, out_ref):
  agg = aggp_ref[0] + aggp_ref[1]
  norm = lax.rsqrt(jnp.maximum(deg_ref[...], 1.0))
  out_ref[...] = jnp.maximum(agg * norm + b1_ref[...], 0.0)


def _out_kernel(h1_ref, nsp_ref, deg_ref, ws_ref, wn_ref, b2_ref, out_ref):
  inv = 1.0 / jnp.maximum(deg_ref[...], 1.0)
  neigh = (nsp_ref[0] + nsp_ref[1]) * inv
  out_ref[...] = (
      jnp.dot(h1_ref[...], ws_ref[...], preferred_element_type=jnp.float32,
              precision=lax.Precision.HIGHEST)
      + jnp.dot(neigh, wn_ref[...], preferred_element_type=jnp.float32,
                precision=lax.Precision.HIGHEST)
      + b2_ref[...])


def kernel(x, edge_index, W1, b1, W_self, W_neigh, b2):
  n, d_in = x.shape
  e = edge_index.shape[1]
  d_hid = W1.shape[1]
  d_out = W_self.shape[1]
  src = edge_index[0]
  dst = edge_index[1]

  n_pad = ((n + 8 * NS) // (8 * NS)) * (8 * NS)
  n2 = n_pad

  import math
  lcm = math.lcm(CHUNK, DEG_CHUNK)
  step = NW * lcm * 8
  e2 = ((e + step - 1) // step) * step
  pad = e2 - e
  pad_lo = jnp.arange(pad, dtype=jnp.int32) % n
  pad_hi = n + jnp.arange(pad, dtype=jnp.int32) % (n_pad - n)
  src_deg = jnp.concatenate([src, pad_hi])
  src_gat = jnp.concatenate([src, pad_lo])
  dst_p = jnp.concatenate([dst, pad_hi])

  deg_parts = _deg_call(src_deg, dst_p, n_pad, e2)
  deg_out_col = (deg_parts[0, :n, 0] + deg_parts[1, :n, 0])[:, None]
  deg_in_col = (deg_parts[0, :n, 1] + deg_parts[1, :n, 1])[:, None]

  grid = n // _BLK
  hs = pl.pallas_call(
      _h_scaled_kernel,
      grid=(grid,),
      in_specs=[
          pl.BlockSpec((_BLK, d_in), lambda i: (i, 0)),
          pl.BlockSpec((d_in, d_hid), lambda i: (0, 0)),
          pl.BlockSpec((_BLK, 1), lambda i: (i, 0)),
      ],
      out_specs=pl.BlockSpec((_BLK, d_hid), lambda i: (i, 0)),
      out_shape=jax.ShapeDtypeStruct((n, d_hid), jnp.float32),
  )(x, W1, deg_out_col)

  agg_parts = _scatter_call(hs, src_gat, dst_p, n2, d_hid, e2)

  h1 = pl.pallas_call(
      _h1_kernel,
      grid=(grid,),
      in_specs=[
          pl.BlockSpec((2, _BLK, d_hid), lambda i: (0, i, 0)),
          pl.BlockSpec((_BLK, 1), lambda i: (i, 0)),
          pl.BlockSpec((d_hid,), lambda i: (0,)),
      ],
      out_specs=pl.BlockSpec((_BLK, d_hid), lambda i: (i, 0)),
      out_shape=jax.ShapeDtypeStruct((n, d_hid), jnp.float32),
  )(agg_parts, deg_in_col, b1)

  ns_parts = _scatter_call(h1, src_gat, dst_p, n2, d_hid, e2)

  out = pl.pallas_call(
      _out_kernel,
      grid=(grid,),
      in_specs=[
          pl.BlockSpec((_BLK, d_hid), lambda i: (i, 0)),
          pl.BlockSpec((2, _BLK, d_hid), lambda i: (0, i, 0)),
          pl.BlockSpec((_BLK, 1), lambda i: (i, 0)),
          pl.BlockSpec((d_hid, d_out), lambda i: (0, 0)),
          pl.BlockSpec((d_hid, d_out), lambda i: (0, 0)),
          pl.BlockSpec((d_out,), lambda i: (0,)),
      ],
      out_specs=pl.BlockSpec((_BLK, d_out), lambda i: (i, 0)),
      out_shape=jax.ShapeDtypeStruct((n, d_out), jnp.float32),
  )(h1, ns_parts, deg_in_col, W_self, W_neigh, b2)

  return out


# speedup vs baseline: 1.2812x; 1.0132x over previous
import functools

import jax
import jax.numpy as jnp
from jax import lax
from jax.experimental import pallas as pl
from jax.experimental.pallas import tpu as pltpu
from jax.experimental.pallas import tpu_sc as plsc

NC = 2
NS = 16
NW = NC * NS
CHUNK = 80
DEG_CHUNK = 128
NBUF = 4
NSLOT = 2 * NBUF


def _mesh():
  return plsc.VectorSubcoreMesh(
      core_axis_name="c", subcore_axis_name="s", num_cores=NC,
      num_subcores=NS)


def _deg_call(src, dst, n_pad, e2):
  epw = e2 // NW
  n_chunks = epw // DEG_CHUNK
  rows_per_tile = n_pad // NS
  zeros = jnp.zeros((rows_per_tile, 2), jnp.float32)
  ones_src = jnp.tile(jnp.array([[1.0, 0.0]], jnp.float32), (DEG_CHUNK, 1))
  ones_dst = jnp.tile(jnp.array([[0.0, 1.0]], jnp.float32), (DEG_CHUNK, 1))

  @functools.partial(
      pl.kernel,
      out_type=jax.ShapeDtypeStruct((NC, n_pad, 2), jnp.float32),
      mesh=_mesh(),
      scratch_types=[
          [pltpu.VMEM((DEG_CHUNK,), jnp.int32) for _ in range(NSLOT)],
          [pltpu.VMEM((DEG_CHUNK,), jnp.int32) for _ in range(NSLOT)],
          pltpu.VMEM((DEG_CHUNK, 2), jnp.float32),
          pltpu.VMEM((DEG_CHUNK, 2), jnp.float32),
          pltpu.VMEM_SHARED((n_pad, 2), jnp.float32),
          [pltpu.SemaphoreType.DMA for _ in range(NSLOT)],
          [pltpu.SemaphoreType.DMA for _ in range(NBUF)],
      ],
  )
  def deg_kernel(src_hbm, dst_hbm, zz_hbm, os_hbm, od_hbm, out_hbm, idx_s,
                 idx_d, ones_s, ones_d, acc, isem, ssem):
    cid = lax.axis_index("c")
    sid = lax.axis_index("s")
    wid = sid * NC + cid
    base0 = wid * epw
    pltpu.sync_copy(zz_hbm, acc.at[pl.ds(sid * rows_per_tile,
                                         rows_per_tile)])
    pltpu.sync_copy(os_hbm, ones_s)
    pltpu.sync_copy(od_hbm, ones_d)
    plsc.subcore_barrier()

    def fire_idx(c, k):
      base = base0 + c * DEG_CHUNK
      pltpu.async_copy(src_hbm.at[pl.ds(base, DEG_CHUNK)], idx_s[k], isem[k])
      pltpu.async_copy(dst_hbm.at[pl.ds(base, DEG_CHUNK)], idx_d[k], isem[k])

    def wait_idx(k):
      pltpu.make_async_copy(src_hbm.at[pl.ds(0, DEG_CHUNK)], idx_s[k],
                            isem[k]).wait()
      pltpu.make_async_copy(dst_hbm.at[pl.ds(0, DEG_CHUNK)], idx_d[k],
                            isem[k]).wait()

    def fire_scatter(k8, k4):
      pltpu.async_copy(ones_s, acc.at[idx_s[k8]], ssem[k4], add=True)
      pltpu.async_copy(ones_d, acc.at[idx_d[k8]], ssem[k4], add=True)

    def wait_scatter(k8, k4):
      pltpu.make_async_copy(ones_s, acc.at[idx_s[k8]], ssem[k4]).wait()
      pltpu.make_async_copy(ones_d, acc.at[idx_d[k8]], ssem[k4]).wait()

    def emit(cpos, k8, do_ws, do_i, do_s):
      if do_ws:
        wait_scatter((k8 - 2) % NSLOT, (k8 - 2) % NBUF)
      if do_i:
        fire_idx(cpos + 4, (k8 + 4) % NSLOT)
      if do_s:
        wait_idx(k8)
        fire_scatter(k8, k8 % NBUF)

    for c in range(4):
      fire_idx(c, c % NSLOT)
    for c in range(8):
      emit(c, c % NSLOT, c >= 2, c + 4 <= n_chunks - 1, True)

    def body(j, _):
      cpos = 8 + j * 8
      for k in range(8):
        emit(cpos + k, k, True, True, True)
      return 0

    lax.fori_loop(0, (n_chunks - 16) // 8, body, 0)
    for c in range(n_chunks - 8, n_chunks + 2):
      emit(c, c % NSLOT, True, c + 4 <= n_chunks - 1, c <= n_chunks - 1)

    plsc.subcore_barrier()
    sl = pl.ds(sid * rows_per_tile, rows_per_tile)
    pltpu.sync_copy(acc.at[sl], out_hbm.at[cid, sl, :])

  return deg_kernel(src, dst, zeros, ones_src, ones_dst)


def _scatter_call(table, src, dst, n2, d, e2):
  epw = e2 // NW
  n_chunks = epw // CHUNK
  rows_per_tile = n2 // NS
  zeros = jnp.zeros((rows_per_tile, d), jnp.float32)

  @functools.partial(
      pl.kernel,
      out_type=jax.ShapeDtypeStruct((NC, n2, d), jnp.float32),
      mesh=_mesh(),
      scratch_types=[
          [pltpu.VMEM((CHUNK,), jnp.int32) for _ in range(NSLOT)],
          [pltpu.VMEM((CHUNK,), jnp.int32) for _ in range(NSLOT)],
          [pltpu.VMEM((CHUNK, d), jnp.float32) for _ in range(NBUF)],
          pltpu.VMEM_SHARED((n2, d), jnp.float32),
          [pltpu.SemaphoreType.DMA for _ in range(NSLOT)],
          [pltpu.SemaphoreType.DMA for _ in range(NBUF)],
          [pltpu.SemaphoreType.DMA for _ in range(NBUF)],
      ],
  )
  def scat_kernel(table_hbm, src_hbm, dst_hbm, zz_hbm, out_hbm, idx_s, idx_d,
                  rows_v, acc, isem, gsem, ssem):
    cid = lax.axis_index("c")
    sid = lax.axis_index("s")
    wid = sid * NC + cid
    base0 = wid * epw
    pltpu.sync_copy(zz_hbm, acc.at[pl.ds(sid * rows_per_tile,
                                         rows_per_tile)])
    plsc.subcore_barrier()

    def fire_idx(c, k):
      base = base0 + c * CHUNK
      pltpu.async_copy(src_hbm.at[pl.ds(base, CHUNK)], idx_s[k], isem[k])
      pltpu.async_copy(dst_hbm.at[pl.ds(base, CHUNK)], idx_d[k], isem[k])

    def wait_idx(k):
      pltpu.make_async_copy(src_hbm.at[pl.ds(0, CHUNK)], idx_s[k],
                            isem[k]).wait()
      pltpu.make_async_copy(dst_hbm.at[pl.ds(0, CHUNK)], idx_d[k],
                            isem[k]).wait()

    def emit(cpos, k8, do_ws, do_i, do_g, do_s):
      k4 = k8 % NBUF
      if do_ws:
        pltpu.make_async_copy(rows_v[k4],
                              acc.at[idx_d[(k8 + 4) % NSLOT]],
                              ssem[k4]).wait()
      if do_i:
        fire_idx(cpos + 4, (k8 + 4) % NSLOT)
      if do_g:
        wait_idx(k8)
        pltpu.async_copy(table_hbm.at[idx_s[k8]], rows_v[k4], gsem[k4])
      if do_s:
        b = (k4 + 2) % NBUF
        s = (k8 + 6) % NSLOT
        pltpu.make_async_copy(table_hbm.at[idx_s[s]], rows_v[b],
                              gsem[b]).wait()
        pltpu.async_copy(rows_v[b], acc.at[idx_d[s]], ssem[b], add=True)

    for c in range(4):
      fire_idx(c, c % NSLOT)
    for c in range(8):
      emit(c, c % NSLOT, c >= 4, c + 4 <= n_chunks - 1, True, c >= 2)

    def body(j, _):
      cpos = 8 + j * 8
      for k in range(8):
        emit(cpos + k, k, True, True, True, True)
      return 0

    lax.fori_loop(0, (n_chunks - 16) // 8, body, 0)
    for c in range(n_chunks - 8, n_chunks + 4):
      emit(c, c % NSLOT, True, c + 4 <= n_chunks - 1, c <= n_chunks - 1,
           c <= n_chunks + 1)

    plsc.subcore_barrier()
    sl = pl.ds(sid * rows_per_tile, rows_per_tile)
    pltpu.sync_copy(acc.at[sl], out_hbm.at[cid, sl])

  return scat_kernel(table, src, dst, zeros)


_BLK = 1000


def _mm_kernel(x_ref, w1_ref, out_ref):
  out_ref[...] = jnp.dot(x_ref[...], w1_ref[...],
                         preferred_element_type=jnp.float32,
                         precision=lax.Precision.HIGHEST)


def _scale_kernel(h_ref, deg_ref, out_ref):
  out_ref[...] = h_ref[...] * lax.rsqrt(jnp.maximum(deg_ref[...], 1.0))


def _h1_kernel(aggp_ref, deg_ref, b1_ref, out_ref):
  agg = aggp_ref[0] + aggp_ref[1]
  norm = lax.rsqrt(jnp.maximum(deg_ref[...], 1.0))
  out_ref[...] = jnp.maximum(agg * norm + b1_ref[...], 0.0)


def _self_kernel(h1_ref, ws_ref, b2_ref, out_ref):
  out_ref[...] = jnp.dot(h1_ref[...], ws_ref[...],
                         preferred_element_type=jnp.float32,
                         precision=lax.Precision.HIGHEST) + b2_ref[...]


def _out_kernel(selfp_ref, nsp_ref, deg_ref, wn_ref, out_ref):
  inv = 1.0 / jnp.maximum(deg_ref[...], 1.0)
  neigh = (nsp_ref[0] + nsp_ref[1]) * inv
  out_ref[...] = selfp_ref[...] + jnp.dot(
      neigh, wn_ref[...], preferred_element_type=jnp.float32,
      precision=lax.Precision.HIGHEST)


def kernel(x, edge_index, W1, b1, W_self, W_neigh, b2):
  n, d_in = x.shape
  e = edge_index.shape[1]
  d_hid = W1.shape[1]
  d_out = W_self.shape[1]
  src = edge_index[0]
  dst = edge_index[1]

  n_pad = ((n + 8 * NS) // (8 * NS)) * (8 * NS)
  n2 = n_pad

  import math
  lcm = math.lcm(CHUNK, DEG_CHUNK)
  step = NW * lcm * 8
  e2 = ((e + step - 1) // step) * step
  pad = e2 - e
  pad_lo = jnp.arange(pad, dtype=jnp.int32) % n
  pad_hi = n + jnp.arange(pad, dtype=jnp.int32) % (n_pad - n)
  src_deg = jnp.concatenate([src, pad_hi])
  src_gat = jnp.concatenate([src, pad_lo])
  dst_p = jnp.concatenate([dst, pad_hi])

  grid = n // _BLK
  h = pl.pallas_call(
      _mm_kernel,
      grid=(grid,),
      in_specs=[
          pl.BlockSpec((_BLK, d_in), lambda i: (i, 0)),
          pl.BlockSpec((d_in, d_hid), lambda i: (0, 0)),
      ],
      out_specs=pl.BlockSpec((_BLK, d_hid), lambda i: (i, 0)),
      out_shape=jax.ShapeDtypeStruct((n, d_hid), jnp.float32),
  )(x, W1)

  deg_parts = _deg_call(src_deg, dst_p, n_pad, e2)
  deg_out_col = (deg_parts[0, :n, 0] + deg_parts[1, :n, 0])[:, None]
  deg_in_col = (deg_parts[0, :n, 1] + deg_parts[1, :n, 1])[:, None]

  hs = pl.pallas_call(
      _scale_kernel,
      grid=(grid,),
      in_specs=[
          pl.BlockSpec((_BLK, d_hid), lambda i: (i, 0)),
          pl.BlockSpec((_BLK, 1), lambda i: (i, 0)),
      ],
      out_specs=pl.BlockSpec((_BLK, d_hid), lambda i: (i, 0)),
      out_shape=jax.ShapeDtypeStruct((n, d_hid), jnp.float32),
  )(h, deg_out_col)

  agg_parts = _scatter_call(hs, src_gat, dst_p, n2, d_hid, e2)

  h1 = pl.pallas_call(
      _h1_kernel,
      grid=(grid,),
      in_specs=[
          pl.BlockSpec((2, _BLK, d_hid), lambda i: (0, i, 0)),
          pl.BlockSpec((_BLK, 1), lambda i: (i, 0)),
          pl.BlockSpec((d_hid,), lambda i: (0,)),
      ],
      out_specs=pl.BlockSpec((_BLK, d_hid), lambda i: (i, 0)),
      out_shape=jax.ShapeDtypeStruct((n, d_hid), jnp.float32),
  )(agg_parts, deg_in_col, b1)

  ns_parts = _scatter_call(h1, src_gat, dst_p, n2, d_hid, e2)

  selfp = pl.pallas_call(
      _self_kernel,
      grid=(grid,),
      in_specs=[
          pl.BlockSpec((_BLK, d_hid), lambda i: (i, 0)),
          pl.BlockSpec((d_hid, d_out), lambda i: (0, 0)),
          pl.BlockSpec((d_out,), lambda i: (0,)),
      ],
      out_specs=pl.BlockSpec((_BLK, d_out), lambda i: (i, 0)),
      out_shape=jax.ShapeDtypeStruct((n, d_out), jnp.float32),
  )(h1, W_self, b2)

  out = pl.pallas_call(
      _out_kernel,
      grid=(grid,),
      in_specs=[
          pl.BlockSpec((_BLK, d_out), lambda i: (i, 0)),
          pl.BlockSpec((2, _BLK, d_hid), lambda i: (0, i, 0)),
          pl.BlockSpec((_BLK, 1), lambda i: (i, 0)),
          pl.BlockSpec((d_hid, d_out), lambda i: (0, 0)),
      ],
      out_specs=pl.BlockSpec((_BLK, d_out), lambda i: (i, 0)),
      out_shape=jax.ShapeDtypeStruct((n, d_out), jnp.float32),
  )(selfp, ns_parts, deg_in_col, W_neigh)

  return out


# speedup vs baseline: 1.3114x; 1.0236x over previous
import functools
import math

import numpy as np

import jax
import jax.numpy as jnp
from jax import lax
from jax.experimental import pallas as pl
from jax.experimental.pallas import tpu as pltpu
from jax.experimental.pallas import tpu_sc as plsc

NC = 2
NS = 16
NW = NC * NS
CHUNK = 80
DEG_CHUNK = 128
NBUF = 4
NSLOT = 2 * NBUF


def _mesh():
  return plsc.VectorSubcoreMesh(
      core_axis_name="c", subcore_axis_name="s", num_cores=NC,
      num_subcores=NS)


def _deg_call(src, dst, n_pad, e2):
  epw = e2 // NW
  n_chunks = epw // DEG_CHUNK
  rows_per_tile = n_pad // NS
  zeros = jnp.zeros((rows_per_tile, 2), jnp.float32)
  ones_src = jnp.tile(jnp.array([[1.0, 0.0]], jnp.float32), (DEG_CHUNK, 1))
  ones_dst = jnp.tile(jnp.array([[0.0, 1.0]], jnp.float32), (DEG_CHUNK, 1))

  @functools.partial(
      pl.kernel,
      out_type=jax.ShapeDtypeStruct((NC, n_pad, 2), jnp.float32),
      mesh=_mesh(),
      scratch_types=[
          [pltpu.VMEM((DEG_CHUNK,), jnp.int32) for _ in range(NSLOT)],
          [pltpu.VMEM((DEG_CHUNK,), jnp.int32) for _ in range(NSLOT)],
          pltpu.VMEM((DEG_CHUNK, 2), jnp.float32),
          pltpu.VMEM((DEG_CHUNK, 2), jnp.float32),
          pltpu.VMEM_SHARED((n_pad, 2), jnp.float32),
          [pltpu.SemaphoreType.DMA for _ in range(NSLOT)],
          [pltpu.SemaphoreType.DMA for _ in range(NBUF)],
      ],
  )
  def deg_kernel(src_hbm, dst_hbm, zz_hbm, os_hbm, od_hbm, out_hbm, idx_s,
                 idx_d, ones_s, ones_d, acc, isem, ssem):
    cid = lax.axis_index("c")
    sid = lax.axis_index("s")
    wid = sid * NC + cid
    base0 = wid * epw
    pltpu.sync_copy(zz_hbm, acc.at[pl.ds(sid * rows_per_tile,
                                         rows_per_tile)])
    pltpu.sync_copy(os_hbm, ones_s)
    pltpu.sync_copy(od_hbm, ones_d)
    plsc.subcore_barrier()

    def fire_idx(c, k):
      base = base0 + c * DEG_CHUNK
      pltpu.async_copy(src_hbm.at[pl.ds(base, DEG_CHUNK)], idx_s[k], isem[k])
      pltpu.async_copy(dst_hbm.at[pl.ds(base, DEG_CHUNK)], idx_d[k], isem[k])

    def wait_idx(k):
      pltpu.make_async_copy(src_hbm.at[pl.ds(0, DEG_CHUNK)], idx_s[k],
                            isem[k]).wait()
      pltpu.make_async_copy(dst_hbm.at[pl.ds(0, DEG_CHUNK)], idx_d[k],
                            isem[k]).wait()

    def fire_scatter(k8, k4):
      pltpu.async_copy(ones_s, acc.at[idx_s[k8]], ssem[k4], add=True)
      pltpu.async_copy(ones_d, acc.at[idx_d[k8]], ssem[k4], add=True)

    def wait_scatter(k8, k4):
      pltpu.make_async_copy(ones_s, acc.at[idx_s[k8]], ssem[k4]).wait()
      pltpu.make_async_copy(ones_d, acc.at[idx_d[k8]], ssem[k4]).wait()

    def emit(cpos, k8, do_ws, do_i, do_s):
      if do_ws:
        wait_scatter((k8 - 2) % NSLOT, (k8 - 2) % NBUF)
      if do_i:
        fire_idx(cpos + 4, (k8 + 4) % NSLOT)
      if do_s:
        wait_idx(k8)
        fire_scatter(k8, k8 % NBUF)

    for c in range(4):
      fire_idx(c, c % NSLOT)
    for c in range(8):
      emit(c, c % NSLOT, c >= 2, c + 4 <= n_chunks - 1, True)

    def body(j, _):
      cpos = 8 + j * 8
      for k in range(8):
        emit(cpos + k, k, True, True, True)
      return 0

    lax.fori_loop(0, (n_chunks - 16) // 8, body, 0)
    for c in range(n_chunks - 8, n_chunks + 2):
      emit(c, c % NSLOT, True, c + 4 <= n_chunks - 1, c <= n_chunks - 1)

    plsc.subcore_barrier()
    sl = pl.ds(sid * rows_per_tile, rows_per_tile)
    pltpu.sync_copy(acc.at[sl], out_hbm.at[cid, sl, :])

  return deg_kernel(src, dst, zeros, ones_src, ones_dst)


def _scatter_call(table, src, dst, n2, d, e2):
  epw = e2 // NW
  n_chunks = epw // CHUNK
  rows_per_tile = n2 // NS
  zeros = jnp.zeros((rows_per_tile, d), jnp.float32)

  @functools.partial(
      pl.kernel,
      out_type=jax.ShapeDtypeStruct((NC, n2, d), jnp.float32),
      mesh=_mesh(),
      scratch_types=[
          [pltpu.VMEM((CHUNK,), jnp.int32) for _ in range(NSLOT)],
          [pltpu.VMEM((CHUNK,), jnp.int32) for _ in range(NSLOT)],
          [pltpu.VMEM((CHUNK, d), jnp.float32) for _ in range(NBUF)],
          pltpu.VMEM_SHARED((n2, d), jnp.float32),
          [pltpu.SemaphoreType.DMA for _ in range(NSLOT)],
          [pltpu.SemaphoreType.DMA for _ in range(NBUF)],
          [pltpu.SemaphoreType.DMA for _ in range(NBUF)],
      ],
  )
  def scat_kernel(table_hbm, src_hbm, dst_hbm, zz_hbm, out_hbm, idx_s, idx_d,
                  rows_v, acc, isem, gsem, ssem):
    cid = lax.axis_index("c")
    sid = lax.axis_index("s")
    wid = sid * NC + cid
    base0 = wid * epw
    pltpu.sync_copy(zz_hbm, acc.at[pl.ds(sid * rows_per_tile,
                                         rows_per_tile)])
    plsc.subcore_barrier()

    def fire_idx(c, k):
      base = base0 + c * CHUNK
      pltpu.async_copy(src_hbm.at[pl.ds(base, CHUNK)], idx_s[k], isem[k])
      pltpu.async_copy(dst_hbm.at[pl.ds(base, CHUNK)], idx_d[k], isem[k])

    def wait_idx(k):
      pltpu.make_async_copy(src_hbm.at[pl.ds(0, CHUNK)], idx_s[k],
                            isem[k]).wait()
      pltpu.make_async_copy(dst_hbm.at[pl.ds(0, CHUNK)], idx_d[k],
                            isem[k]).wait()

    def emit(cpos, k8, do_ws, do_i, do_g, do_s):
      k4 = k8 % NBUF
      if do_ws:
        pltpu.make_async_copy(rows_v[k4],
                              acc.at[idx_d[(k8 + 4) % NSLOT]],
                              ssem[k4]).wait()
      if do_i:
        fire_idx(cpos + 4, (k8 + 4) % NSLOT)
      if do_g:
        wait_idx(k8)
        pltpu.async_copy(table_hbm.at[idx_s[k8]], rows_v[k4], gsem[k4])
      if do_s:
        b = (k4 + 2) % NBUF
        s = (k8 + 6) % NSLOT
        pltpu.make_async_copy(table_hbm.at[idx_s[s]], rows_v[b],
                              gsem[b]).wait()
        pltpu.async_copy(rows_v[b], acc.at[idx_d[s]], ssem[b], add=True)

    for c in range(4):
      fire_idx(c, c % NSLOT)
    for c in range(8):
      emit(c, c % NSLOT, c >= 4, c + 4 <= n_chunks - 1, True, c >= 2)

    def body(j, _):
      cpos = 8 + j * 8
      for k in range(8):
        emit(cpos + k, k, True, True, True, True)
      return 0

    lax.fori_loop(0, (n_chunks - 16) // 8, body, 0)
    for c in range(n_chunks - 8, n_chunks + 4):
      emit(c, c % NSLOT, True, c + 4 <= n_chunks - 1, c <= n_chunks - 1,
           c <= n_chunks + 1)

    plsc.subcore_barrier()
    sl = pl.ds(sid * rows_per_tile, rows_per_tile)
    pltpu.sync_copy(acc.at[sl], out_hbm.at[cid, sl])

  return scat_kernel(table, src, dst, zeros)


_BLK = 1000


def _mm_kernel(x_ref, w1_ref, out_ref):
  out_ref[...] = jnp.dot(x_ref[...], w1_ref[...],
                         preferred_element_type=jnp.float32,
                         precision=lax.Precision.HIGHEST)


def _scale_kernel(h_ref, degp_ref, out_ref):
  deg = degp_ref[0, :, 0] + degp_ref[1, :, 0]
  out_ref[...] = h_ref[...] * lax.rsqrt(jnp.maximum(deg, 1.0))[:, None]


def _h1_kernel(aggp_ref, degp_ref, b1_ref, out_ref):
  agg = aggp_ref[0] + aggp_ref[1]
  deg = degp_ref[0, :, 1] + degp_ref[1, :, 1]
  norm = lax.rsqrt(jnp.maximum(deg, 1.0))[:, None]
  out_ref[...] = jnp.maximum(agg * norm + b1_ref[...], 0.0)


def _self_kernel(h1_ref, ws_ref, b2_ref, out_ref):
  out_ref[...] = jnp.dot(h1_ref[...], ws_ref[...],
                         preferred_element_type=jnp.float32,
                         precision=lax.Precision.HIGHEST) + b2_ref[...]


def _out_kernel(selfp_ref, nsp_ref, degp_ref, wn_ref, out_ref):
  deg = degp_ref[0, :, 1] + degp_ref[1, :, 1]
  inv = (1.0 / jnp.maximum(deg, 1.0))[:, None]
  neigh = (nsp_ref[0] + nsp_ref[1]) * inv
  out_ref[...] = selfp_ref[...] + jnp.dot(
      neigh, wn_ref[...], preferred_element_type=jnp.float32,
      precision=lax.Precision.HIGHEST)


def kernel(x, edge_index, W1, b1, W_self, W_neigh, b2):
  n, d_in = x.shape
  e = edge_index.shape[1]
  d_hid = W1.shape[1]
  d_out = W_self.shape[1]
  src = edge_index[0]
  dst = edge_index[1]

  n_pad = ((n + 8 * NS) // (8 * NS)) * (8 * NS)
  n2 = n_pad

  lcm = math.lcm(CHUNK, DEG_CHUNK)
  step = NW * lcm * 8
  e2 = ((e + step - 1) // step) * step
  pad = e2 - e
  pad_lo = jnp.asarray(np.arange(pad, dtype=np.int32) % n)
  pad_hi = jnp.asarray(n + np.arange(pad, dtype=np.int32) % (n_pad - n))
  src_deg = jnp.concatenate([src, pad_hi])
  src_gat = jnp.concatenate([src, pad_lo])
  dst_p = jnp.concatenate([dst, pad_hi])

  grid = n // _BLK
  h = pl.pallas_call(
      _mm_kernel,
      grid=(grid,),
      in_specs=[
          pl.BlockSpec((_BLK, d_in), lambda i: (i, 0)),
          pl.BlockSpec((d_in, d_hid), lambda i: (0, 0)),
      ],
      out_specs=pl.BlockSpec((_BLK, d_hid), lambda i: (i, 0)),
      out_shape=jax.ShapeDtypeStruct((n, d_hid), jnp.float32),
  )(x, W1)

  deg_parts = _deg_call(src_deg, dst_p, n_pad, e2)

  hs = pl.pallas_call(
      _scale_kernel,
      grid=(grid,),
      in_specs=[
          pl.BlockSpec((_BLK, d_hid), lambda i: (i, 0)),
          pl.BlockSpec((2, _BLK, 2), lambda i: (0, i, 0)),
      ],
      out_specs=pl.BlockSpec((_BLK, d_hid), lambda i: (i, 0)),
      out_shape=jax.ShapeDtypeStruct((n, d_hid), jnp.float32),
  )(h, deg_parts)

  agg_parts = _scatter_call(hs, src_gat, dst_p, n2, d_hid, e2)

  h1 = pl.pallas_call(
      _h1_kernel,
      grid=(grid,),
      in_specs=[
          pl.BlockSpec((2, _BLK, d_hid), lambda i: (0, i, 0)),
          pl.BlockSpec((2, _BLK, 2), lambda i: (0, i, 0)),
          pl.BlockSpec((d_hid,), lambda i: (0,)),
      ],
      out_specs=pl.BlockSpec((_BLK, d_hid), lambda i: (i, 0)),
      out_shape=jax.ShapeDtypeStruct((n, d_hid), jnp.float32),
  )(agg_parts, deg_parts, b1)

  ns_parts = _scatter_call(h1, src_gat, dst_p, n2, d_hid, e2)

  selfp = pl.pallas_call(
      _self_kernel,
      grid=(grid,),
      in_specs=[
          pl.BlockSpec((_BLK, d_hid), lambda i: (i, 0)),
          pl.BlockSpec((d_hid, d_out), lambda i: (0, 0)),
          pl.BlockSpec((d_out,), lambda i: (0,)),
      ],
      out_specs=pl.BlockSpec((_BLK, d_out), lambda i: (i, 0)),
      out_shape=jax.ShapeDtypeStruct((n, d_out), jnp.float32),
  )(h1, W_self, b2)

  out = pl.pallas_call(
      _out_kernel,
      grid=(grid,),
      in_specs=[
          pl.BlockSpec((_BLK, d_out), lambda i: (i, 0)),
          pl.BlockSpec((2, _BLK, d_hid), lambda i: (0, i, 0)),
          pl.BlockSpec((2, _BLK, 2), lambda i: (0, i, 0)),
          pl.BlockSpec((d_hid, d_out), lambda i: (0, 0)),
      ],
      out_specs=pl.BlockSpec((_BLK, d_out), lambda i: (i, 0)),
      out_shape=jax.ShapeDtypeStruct((n, d_out), jnp.float32),
  )(selfp, ns_parts, deg_parts, W_neigh)

  return out


# speedup vs baseline: 1.3386x; 1.0207x over previous
import functools
import math

import numpy as np

import jax
import jax.numpy as jnp
from jax import lax
from jax.experimental import pallas as pl
from jax.experimental.pallas import tpu as pltpu
from jax.experimental.pallas import tpu_sc as plsc

NC = 2
NS = 16
NW = NC * NS
CHUNK = 80
DEG_CHUNK = 128
NBUF = 4
NSLOT = 2 * NBUF


def _mesh():
  return plsc.VectorSubcoreMesh(
      core_axis_name="c", subcore_axis_name="s", num_cores=NC,
      num_subcores=NS)


def _deg_call(src, dst, n_pad, e2):
  epw = e2 // NW
  n_chunks = epw // DEG_CHUNK
  rows_per_tile = n_pad // NS
  zeros = jnp.zeros((rows_per_tile, 2), jnp.float32)
  ones_src = jnp.tile(jnp.array([[1.0, 0.0]], jnp.float32), (DEG_CHUNK, 1))
  ones_dst = jnp.tile(jnp.array([[0.0, 1.0]], jnp.float32), (DEG_CHUNK, 1))

  @functools.partial(
      pl.kernel,
      out_type=jax.ShapeDtypeStruct((NC, n_pad, 2), jnp.float32),
      mesh=_mesh(),
      scratch_types=[
          [pltpu.VMEM((DEG_CHUNK,), jnp.int32) for _ in range(NSLOT)],
          [pltpu.VMEM((DEG_CHUNK,), jnp.int32) for _ in range(NSLOT)],
          pltpu.VMEM((DEG_CHUNK, 2), jnp.float32),
          pltpu.VMEM((DEG_CHUNK, 2), jnp.float32),
          pltpu.VMEM_SHARED((n_pad, 2), jnp.float32),
          [pltpu.SemaphoreType.DMA for _ in range(NSLOT)],
          [pltpu.SemaphoreType.DMA for _ in range(NBUF)],
      ],
  )
  def deg_kernel(src_hbm, dst_hbm, zz_hbm, os_hbm, od_hbm, out_hbm, idx_s,
                 idx_d, ones_s, ones_d, acc, isem, ssem):
    cid = lax.axis_index("c")
    sid = lax.axis_index("s")
    wid = sid * NC + cid
    base0 = wid * epw
    pltpu.sync_copy(zz_hbm, acc.at[pl.ds(sid * rows_per_tile,
                                         rows_per_tile)])
    pltpu.sync_copy(os_hbm, ones_s)
    pltpu.sync_copy(od_hbm, ones_d)
    plsc.subcore_barrier()

    def fire_idx(c, k):
      base = base0 + c * DEG_CHUNK
      pltpu.async_copy(src_hbm.at[pl.ds(base, DEG_CHUNK)], idx_s[k], isem[k])
      pltpu.async_copy(dst_hbm.at[pl.ds(base, DEG_CHUNK)], idx_d[k], isem[k])

    def wait_idx(k):
      pltpu.make_async_copy(src_hbm.at[pl.ds(0, DEG_CHUNK)], idx_s[k],
                            isem[k]).wait()
      pltpu.make_async_copy(dst_hbm.at[pl.ds(0, DEG_CHUNK)], idx_d[k],
                            isem[k]).wait()

    def fire_scatter(k8, k4):
      pltpu.async_copy(ones_s, acc.at[idx_s[k8]], ssem[k4], add=True)
      pltpu.async_copy(ones_d, acc.at[idx_d[k8]], ssem[k4], add=True)

    def wait_scatter(k8, k4):
      pltpu.make_async_copy(ones_s, acc.at[idx_s[k8]], ssem[k4]).wait()
      pltpu.make_async_copy(ones_d, acc.at[idx_d[k8]], ssem[k4]).wait()

    def emit(cpos, k8, do_ws, do_i, do_s):
      if do_ws:
        wait_scatter((k8 - 2) % NSLOT, (k8 - 2) % NBUF)
      if do_i:
        fire_idx(cpos + 4, (k8 + 4) % NSLOT)
      if do_s:
        wait_idx(k8)
        fire_scatter(k8, k8 % NBUF)

    for c in range(4):
      fire_idx(c, c % NSLOT)
    for c in range(8):
      emit(c, c % NSLOT, c >= 2, c + 4 <= n_chunks - 1, True)

    def body(j, _):
      cpos = 8 + j * 8
      for k in range(8):
        emit(cpos + k, k, True, True, True)
      return 0

    lax.fori_loop(0, (n_chunks - 16) // 8, body, 0)
    for c in range(n_chunks - 8, n_chunks + 2):
      emit(c, c % NSLOT, True, c + 4 <= n_chunks - 1, c <= n_chunks - 1)

    plsc.subcore_barrier()
    sl = pl.ds(sid * rows_per_tile, rows_per_tile)
    pltpu.sync_copy(acc.at[sl], out_hbm.at[cid, sl, :])

  return deg_kernel(src, dst, zeros, ones_src, ones_dst)


def _scatter_call(table, src, dst, n2, d, e2):
  epw = e2 // NW
  n_chunks = epw // CHUNK
  rows_per_tile = n2 // NS
  zeros = jnp.zeros((rows_per_tile, d), jnp.float32)

  @functools.partial(
      pl.kernel,
      out_type=jax.ShapeDtypeStruct((NC, n2, d), jnp.float32),
      mesh=_mesh(),
      scratch_types=[
          [pltpu.VMEM((CHUNK,), jnp.int32) for _ in range(NSLOT)],
          [pltpu.VMEM((CHUNK,), jnp.int32) for _ in range(NSLOT)],
          [pltpu.VMEM((CHUNK, d), jnp.float32) for _ in range(NBUF)],
          pltpu.VMEM_SHARED((n2, d), jnp.float32),
          [pltpu.SemaphoreType.DMA for _ in range(NSLOT)],
          [pltpu.SemaphoreType.DMA for _ in range(NBUF)],
          [pltpu.SemaphoreType.DMA for _ in range(NBUF)],
      ],
  )
  def scat_kernel(table_hbm, src_hbm, dst_hbm, zz_hbm, out_hbm, idx_s, idx_d,
                  rows_v, acc, isem, gsem, ssem):
    cid = lax.axis_index("c")
    sid = lax.axis_index("s")
    wid = sid * NC + cid
    base0 = wid * epw
    pltpu.sync_copy(zz_hbm, acc.at[pl.ds(sid * rows_per_tile,
                                         rows_per_tile)])
    plsc.subcore_barrier()

    def fire_idx(c, k):
      base = base0 + c * CHUNK
      pltpu.async_copy(src_hbm.at[pl.ds(base, CHUNK)], idx_s[k], isem[k])
      pltpu.async_copy(dst_hbm.at[pl.ds(base, CHUNK)], idx_d[k], isem[k])

    def wait_idx(k):
      pltpu.make_async_copy(src_hbm.at[pl.ds(0, CHUNK)], idx_s[k],
                            isem[k]).wait()
      pltpu.make_async_copy(dst_hbm.at[pl.ds(0, CHUNK)], idx_d[k],
                            isem[k]).wait()

    def emit(cpos, k8, do_ws, do_i, do_g, do_s):
      k4 = k8 % NBUF
      if do_ws:
        pltpu.make_async_copy(rows_v[k4],
                              acc.at[idx_d[(k8 + 4) % NSLOT]],
                              ssem[k4]).wait()
      if do_i:
        fire_idx(cpos + 4, (k8 + 4) % NSLOT)
      if do_g:
        wait_idx(k8)
        pltpu.async_copy(table_hbm.at[idx_s[k8]], rows_v[k4], gsem[k4])
      if do_s:
        b = (k4 + 2) % NBUF
        s = (k8 + 6) % NSLOT
        pltpu.make_async_copy(table_hbm.at[idx_s[s]], rows_v[b],
                              gsem[b]).wait()
        pltpu.async_copy(rows_v[b], acc.at[idx_d[s]], ssem[b], add=True)

    for c in range(4):
      fire_idx(c, c % NSLOT)
    for c in range(8):
      emit(c, c % NSLOT, c >= 4, c + 4 <= n_chunks - 1, True, c >= 2)

    def body(j, _):
      cpos = 8 + j * 8
      for k in range(8):
        emit(cpos + k, k, True, True, True, True)
      return 0

    lax.fori_loop(0, (n_chunks - 16) // 8, body, 0)
    for c in range(n_chunks - 8, n_chunks + 4):
      emit(c, c % NSLOT, True, c + 4 <= n_chunks - 1, c <= n_chunks - 1,
           c <= n_chunks + 1)

    plsc.subcore_barrier()
    sl = pl.ds(sid * rows_per_tile, rows_per_tile)
    pltpu.sync_copy(acc.at[sl], out_hbm.at[cid, sl])

  return scat_kernel(table, src, dst, zeros)


_BLK = 2000


def _mm_kernel(x_ref, w1_ref, out_ref):
  out_ref[...] = jnp.dot(x_ref[...], w1_ref[...],
                         preferred_element_type=jnp.float32,
                         precision=lax.Precision.HIGHEST)


def _scale_kernel(h_ref, degp_ref, out_ref):
  deg = degp_ref[0, :, 0] + degp_ref[1, :, 0]
  out_ref[...] = h_ref[...] * lax.rsqrt(jnp.maximum(deg, 1.0))[:, None]


def _h1_kernel(aggp_ref, degp_ref, b1_ref, out_ref):
  agg = aggp_ref[0] + aggp_ref[1]
  deg = degp_ref[0, :, 1] + degp_ref[1, :, 1]
  norm = lax.rsqrt(jnp.maximum(deg, 1.0))[:, None]
  out_ref[...] = jnp.maximum(agg * norm + b1_ref[...], 0.0)


def _self_kernel(h1_ref, ws_ref, b2_ref, out_ref):
  out_ref[...] = jnp.dot(h1_ref[...], ws_ref[...],
                         preferred_element_type=jnp.float32,
                         precision=lax.Precision.HIGHEST) + b2_ref[...]


def _out_kernel(selfp_ref, nsp_ref, degp_ref, wn_ref, out_ref):
  deg = degp_ref[0, :, 1] + degp_ref[1, :, 1]
  inv = (1.0 / jnp.maximum(deg, 1.0))[:, None]
  neigh = (nsp_ref[0] + nsp_ref[1]) * inv
  out_ref[...] = selfp_ref[...] + jnp.dot(
      neigh, wn_ref[...], preferred_element_type=jnp.float32,
      precision=lax.Precision.HIGHEST)


def kernel(x, edge_index, W1, b1, W_self, W_neigh, b2):
  n, d_in = x.shape
  e = edge_index.shape[1]
  d_hid = W1.shape[1]
  d_out = W_self.shape[1]
  src = edge_index[0]
  dst = edge_index[1]

  n_pad = ((n + 8 * NS) // (8 * NS)) * (8 * NS)
  n2 = n_pad

  lcm = math.lcm(CHUNK, DEG_CHUNK)
  step = NW * lcm * 8
  e2 = ((e + step - 1) // step) * step
  pad = e2 - e
  pad_lo = jnp.asarray(np.arange(pad, dtype=np.int32) % n)
  pad_hi = jnp.asarray(n + np.arange(pad, dtype=np.int32) % (n_pad - n))
  src_deg = jnp.concatenate([src, pad_hi])
  src_gat = jnp.concatenate([src, pad_lo])
  dst_p = jnp.concatenate([dst, pad_hi])

  grid = n // _BLK
  h = pl.pallas_call(
      _mm_kernel,
      grid=(grid,),
      in_specs=[
          pl.BlockSpec((_BLK, d_in), lambda i: (i, 0)),
          pl.BlockSpec((d_in, d_hid), lambda i: (0, 0)),
      ],
      out_specs=pl.BlockSpec((_BLK, d_hid), lambda i: (i, 0)),
      out_shape=jax.ShapeDtypeStruct((n, d_hid), jnp.float32),
  )(x, W1)

  deg_parts = _deg_call(src_deg, dst_p, n_pad, e2)

  hs = pl.pallas_call(
      _scale_kernel,
      grid=(grid,),
      in_specs=[
          pl.BlockSpec((_BLK, d_hid), lambda i: (i, 0)),
          pl.BlockSpec((2, _BLK, 2), lambda i: (0, i, 0)),
      ],
      out_specs=pl.BlockSpec((_BLK, d_hid), lambda i: (i, 0)),
      out_shape=jax.ShapeDtypeStruct((n, d_hid), jnp.float32),
  )(h, deg_parts)

  agg_parts = _scatter_call(hs, src_gat, dst_p, n2, d_hid, e2)

  h1 = pl.pallas_call(
      _h1_kernel,
      grid=(grid,),
      in_specs=[
          pl.BlockSpec((2, _BLK, d_hid), lambda i: (0, i, 0)),
          pl.BlockSpec((2, _BLK, 2), lambda i: (0, i, 0)),
          pl.BlockSpec((d_hid,), lambda i: (0,)),
      ],
      out_specs=pl.BlockSpec((_BLK, d_hid), lambda i: (i, 0)),
      out_shape=jax.ShapeDtypeStruct((n, d_hid), jnp.float32),
  )(agg_parts, deg_parts, b1)

  ns_parts = _scatter_call(h1, src_gat, dst_p, n2, d_hid, e2)

  selfp = pl.pallas_call(
      _self_kernel,
      grid=(grid,),
      in_specs=[
          pl.BlockSpec((_BLK, d_hid), lambda i: (i, 0)),
          pl.BlockSpec((d_hid, d_out), lambda i: (0, 0)),
          pl.BlockSpec((d_out,), lambda i: (0,)),
      ],
      out_specs=pl.BlockSpec((_BLK, d_out), lambda i: (i, 0)),
      out_shape=jax.ShapeDtypeStruct((n, d_out), jnp.float32),
  )(h1, W_self, b2)

  out = pl.pallas_call(
      _out_kernel,
      grid=(grid,),
      in_specs=[
          pl.BlockSpec((_BLK, d_out), lambda i: (i, 0)),
          pl.BlockSpec((2, _BLK, d_hid), lambda i: (0, i, 0)),
          pl.BlockSpec((2, _BLK, 2), lambda i: (0, i, 0)),
          pl.BlockSpec((d_hid, d_out), lambda i: (0, 0)),
      ],
      out_specs=pl.BlockSpec((_BLK, d_out), lambda i: (i, 0)),
      out_shape=jax.ShapeDtypeStruct((n, d_out), jnp.float32),
  )(selfp, ns_parts, deg_parts, W_neigh)

  return out


# speedup vs baseline: 1.3390x; 1.0003x over previous
import functools
import math

import numpy as np

import jax
import jax.numpy as jnp
from jax import lax
from jax.experimental import pallas as pl
from jax.experimental.pallas import tpu as pltpu
from jax.experimental.pallas import tpu_sc as plsc

NC = 2
NS = 16
NW = NC * NS
CHUNK = 80
DEG_CHUNK = 128
NBUF = 4
NSLOT = 2 * NBUF


def _mesh():
  return plsc.VectorSubcoreMesh(
      core_axis_name="c", subcore_axis_name="s", num_cores=NC,
      num_subcores=NS)


def _deg_call(src, dst, n_pad, e2):
  epw = e2 // NW
  n_chunks = epw // DEG_CHUNK
  rows_per_tile = n_pad // NS
  zeros = jnp.zeros((rows_per_tile, 2), jnp.float32)
  ones_src = jnp.tile(jnp.array([[1.0, 0.0]], jnp.float32), (DEG_CHUNK, 1))
  ones_dst = jnp.tile(jnp.array([[0.0, 1.0]], jnp.float32), (DEG_CHUNK, 1))

  @functools.partial(
      pl.kernel,
      out_type=jax.ShapeDtypeStruct((NC, n_pad, 2), jnp.float32),
      mesh=_mesh(),
      scratch_types=[
          [pltpu.VMEM((DEG_CHUNK,), jnp.int32) for _ in range(NSLOT)],
          [pltpu.VMEM((DEG_CHUNK,), jnp.int32) for _ in range(NSLOT)],
          pltpu.VMEM((DEG_CHUNK, 2), jnp.float32),
          pltpu.VMEM((DEG_CHUNK, 2), jnp.float32),
          pltpu.VMEM_SHARED((n_pad, 2), jnp.float32),
          [pltpu.SemaphoreType.DMA for _ in range(NSLOT)],
          [pltpu.SemaphoreType.DMA for _ in range(NBUF)],
      ],
  )
  def deg_kernel(src_hbm, dst_hbm, zz_hbm, os_hbm, od_hbm, out_hbm, idx_s,
                 idx_d, ones_s, ones_d, acc, isem, ssem):
    cid = lax.axis_index("c")
    sid = lax.axis_index("s")
    wid = sid * NC + cid
    base0 = wid * epw
    pltpu.sync_copy(zz_hbm, acc.at[pl.ds(sid * rows_per_tile,
                                         rows_per_tile)])
    pltpu.sync_copy(os_hbm, ones_s)
    pltpu.sync_copy(od_hbm, ones_d)
    plsc.subcore_barrier()

    def fire_idx(c, k):
      base = base0 + c * DEG_CHUNK
      pltpu.async_copy(src_hbm.at[pl.ds(base, DEG_CHUNK)], idx_s[k], isem[k])
      pltpu.async_copy(dst_hbm.at[pl.ds(base, DEG_CHUNK)], idx_d[k], isem[k])

    def wait_idx(k):
      pltpu.make_async_copy(src_hbm.at[pl.ds(0, DEG_CHUNK)], idx_s[k],
                            isem[k]).wait()
      pltpu.make_async_copy(dst_hbm.at[pl.ds(0, DEG_CHUNK)], idx_d[k],
                            isem[k]).wait()

    def fire_scatter(k8, k4):
      pltpu.async_copy(ones_s, acc.at[idx_s[k8]], ssem[k4], add=True)
      pltpu.async_copy(ones_d, acc.at[idx_d[k8]], ssem[k4], add=True)

    def wait_scatter(k8, k4):
      pltpu.make_async_copy(ones_s, acc.at[idx_s[k8]], ssem[k4]).wait()
      pltpu.make_async_copy(ones_d, acc.at[idx_d[k8]], ssem[k4]).wait()

    def emit(cpos, k8, do_ws, do_i, do_s):
      if do_ws:
        wait_scatter((k8 - 2) % NSLOT, (k8 - 2) % NBUF)
      if do_i:
        fire_idx(cpos + 4, (k8 + 4) % NSLOT)
      if do_s:
        wait_idx(k8)
        fire_scatter(k8, k8 % NBUF)

    for c in range(4):
      fire_idx(c, c % NSLOT)
    for c in range(8):
      emit(c, c % NSLOT, c >= 2, c + 4 <= n_chunks - 1, True)

    def body(j, _):
      cpos = 8 + j * 8
      for k in range(8):
        emit(cpos + k, k, True, True, True)
      return 0

    lax.fori_loop(0, (n_chunks - 16) // 8, body, 0)
    for c in range(n_chunks - 8, n_chunks + 2):
      emit(c, c % NSLOT, True, c + 4 <= n_chunks - 1, c <= n_chunks - 1)

    plsc.subcore_barrier()
    sl = pl.ds(sid * rows_per_tile, rows_per_tile)
    pltpu.sync_copy(acc.at[sl], out_hbm.at[cid, sl, :])

  return deg_kernel(src, dst, zeros, ones_src, ones_dst)


def _scatter_call(table, src, dst, n2, d, e2):
  epw = e2 // NW
  n_chunks = epw // CHUNK
  rows_per_tile = n2 // NS
  zeros = jnp.zeros((rows_per_tile, d), jnp.float32)

  @functools.partial(
      pl.kernel,
      out_type=jax.ShapeDtypeStruct((NC, n2, d), jnp.float32),
      mesh=_mesh(),
      scratch_types=[
          [pltpu.VMEM((CHUNK,), jnp.int32) for _ in range(NSLOT)],
          [pltpu.VMEM((CHUNK,), jnp.int32) for _ in range(NSLOT)],
          [pltpu.VMEM((CHUNK, d), jnp.float32) for _ in range(NBUF)],
          pltpu.VMEM_SHARED((n2, d), jnp.float32),
          [pltpu.SemaphoreType.DMA for _ in range(NSLOT)],
          [pltpu.SemaphoreType.DMA for _ in range(NBUF)],
          [pltpu.SemaphoreType.DMA for _ in range(NBUF)],
      ],
  )
  def scat_kernel(table_hbm, src_hbm, dst_hbm, zz_hbm, out_hbm, idx_s, idx_d,
                  rows_v, acc, isem, gsem, ssem):
    cid = lax.axis_index("c")
    sid = lax.axis_index("s")
    wid = sid * NC + cid
    base0 = wid * epw
    pltpu.sync_copy(zz_hbm, acc.at[pl.ds(sid * rows_per_tile,
                                         rows_per_tile)])
    plsc.subcore_barrier()

    def fire_idx(c, k):
      base = base0 + c * CHUNK
      pltpu.async_copy(src_hbm.at[pl.ds(base, CHUNK)], idx_s[k], isem[k])
      pltpu.async_copy(dst_hbm.at[pl.ds(base, CHUNK)], idx_d[k], isem[k])

    def wait_idx(k):
      pltpu.make_async_copy(src_hbm.at[pl.ds(0, CHUNK)], idx_s[k],
                            isem[k]).wait()
      pltpu.make_async_copy(dst_hbm.at[pl.ds(0, CHUNK)], idx_d[k],
                            isem[k]).wait()

    def emit(cpos, k8, do_ws, do_i, do_g, do_s):
      k4 = k8 % NBUF
      if do_ws:
        pltpu.make_async_copy(rows_v[k4],
                              acc.at[idx_d[(k8 + 4) % NSLOT]],
                              ssem[k4]).wait()
      if do_i:
        fire_idx(cpos + 4, (k8 + 4) % NSLOT)
      if do_g:
        wait_idx(k8)
        pltpu.async_copy(table_hbm.at[idx_s[k8]], rows_v[k4], gsem[k4])
      if do_s:
        b = (k4 + 2) % NBUF
        s = (k8 + 6) % NSLOT
        pltpu.make_async_copy(table_hbm.at[idx_s[s]], rows_v[b],
                              gsem[b]).wait()
        pltpu.async_copy(rows_v[b], acc.at[idx_d[s]], ssem[b], add=True)

    for c in range(4):
      fire_idx(c, c % NSLOT)
    for c in range(8):
      emit(c, c % NSLOT, c >= 4, c + 4 <= n_chunks - 1, True, c >= 2)

    def body(j, _):
      cpos = 8 + j * 8
      for k in range(8):
        emit(cpos + k, k, True, True, True, True)
      return 0

    lax.fori_loop(0, (n_chunks - 16) // 8, body, 0)
    for c in range(n_chunks - 8, n_chunks + 4):
      emit(c, c % NSLOT, True, c + 4 <= n_chunks - 1, c <= n_chunks - 1,
           c <= n_chunks + 1)

    plsc.subcore_barrier()
    sl = pl.ds(sid * rows_per_tile, rows_per_tile)
    pltpu.sync_copy(acc.at[sl], out_hbm.at[cid, sl])

  return scat_kernel(table, src, dst, zeros)


_BLK = 5000


def _mm_kernel(x_ref, w1_ref, out_ref):
  out_ref[...] = jnp.dot(x_ref[...], w1_ref[...],
                         preferred_element_type=jnp.float32,
                         precision=lax.Precision.HIGHEST)


def _scale_kernel(h_ref, degp_ref, out_ref):
  deg = degp_ref[0, :, 0] + degp_ref[1, :, 0]
  out_ref[...] = h_ref[...] * lax.rsqrt(jnp.maximum(deg, 1.0))[:, None]


def _h1_kernel(aggp_ref, degp_ref, b1_ref, out_ref):
  agg = aggp_ref[0] + aggp_ref[1]
  deg = degp_ref[0, :, 1] + degp_ref[1, :, 1]
  norm = lax.rsqrt(jnp.maximum(deg, 1.0))[:, None]
  out_ref[...] = jnp.maximum(agg * norm + b1_ref[...], 0.0)


def _self_kernel(h1_ref, ws_ref, b2_ref, out_ref):
  out_ref[...] = jnp.dot(h1_ref[...], ws_ref[...],
                         preferred_element_type=jnp.float32,
                         precision=lax.Precision.HIGHEST) + b2_ref[...]


def _out_kernel(selfp_ref, nsp_ref, degp_ref, wn_ref, out_ref):
  deg = degp_ref[0, :, 1] + degp_ref[1, :, 1]
  inv = (1.0 / jnp.maximum(deg, 1.0))[:, None]
  neigh = (nsp_ref[0] + nsp_ref[1]) * inv
  out_ref[...] = selfp_ref[...] + jnp.dot(
      neigh, wn_ref[...], preferred_element_type=jnp.float32,
      precision=lax.Precision.HIGHEST)


def kernel(x, edge_index, W1, b1, W_self, W_neigh, b2):
  n, d_in = x.shape
  e = edge_index.shape[1]
  d_hid = W1.shape[1]
  d_out = W_self.shape[1]
  src = edge_index[0]
  dst = edge_index[1]

  n_pad = ((n + 8 * NS) // (8 * NS)) * (8 * NS)
  n2 = n_pad

  lcm = math.lcm(CHUNK, DEG_CHUNK)
  step = NW * lcm * 8
  e2 = ((e + step - 1) // step) * step
  pad = e2 - e
  pad_lo = jnp.asarray(np.arange(pad, dtype=np.int32) % n)
  pad_hi = jnp.asarray(n + np.arange(pad, dtype=np.int32) % (n_pad - n))
  src_deg = jnp.concatenate([src, pad_hi])
  src_gat = jnp.concatenate([src, pad_lo])
  dst_p = jnp.concatenate([dst, pad_hi])

  grid = n // _BLK
  h = pl.pallas_call(
      _mm_kernel,
      grid=(grid,),
      in_specs=[
          pl.BlockSpec((_BLK, d_in), lambda i: (i, 0)),
          pl.BlockSpec((d_in, d_hid), lambda i: (0, 0)),
      ],
      out_specs=pl.BlockSpec((_BLK, d_hid), lambda i: (i, 0)),
      out_shape=jax.ShapeDtypeStruct((n, d_hid), jnp.float32),
  )(x, W1)

  deg_parts = _deg_call(src_deg, dst_p, n_pad, e2)

  hs = pl.pallas_call(
      _scale_kernel,
      grid=(grid,),
      in_specs=[
          pl.BlockSpec((_BLK, d_hid), lambda i: (i, 0)),
          pl.BlockSpec((2, _BLK, 2), lambda i: (0, i, 0)),
      ],
      out_specs=pl.BlockSpec((_BLK, d_hid), lambda i: (i, 0)),
      out_shape=jax.ShapeDtypeStruct((n, d_hid), jnp.float32),
  )(h, deg_parts)

  agg_parts = _scatter_call(hs, src_gat, dst_p, n2, d_hid, e2)

  h1 = pl.pallas_call(
      _h1_kernel,
      grid=(grid,),
      in_specs=[
          pl.BlockSpec((2, _BLK, d_hid), lambda i: (0, i, 0)),
          pl.BlockSpec((2, _BLK, 2), lambda i: (0, i, 0)),
          pl.BlockSpec((d_hid,), lambda i: (0,)),
      ],
      out_specs=pl.BlockSpec((_BLK, d_hid), lambda i: (i, 0)),
      out_shape=jax.ShapeDtypeStruct((n, d_hid), jnp.float32),
  )(agg_parts, deg_parts, b1)

  ns_parts = _scatter_call(h1, src_gat, dst_p, n2, d_hid, e2)

  selfp = pl.pallas_call(
      _self_kernel,
      grid=(grid,),
      in_specs=[
          pl.BlockSpec((_BLK, d_hid), lambda i: (i, 0)),
          pl.BlockSpec((d_hid, d_out), lambda i: (0, 0)),
          pl.BlockSpec((d_out,), lambda i: (0,)),
      ],
      out_specs=pl.BlockSpec((_BLK, d_out), lambda i: (i, 0)),
      out_shape=jax.ShapeDtypeStruct((n, d_out), jnp.float32),
  )(h1, W_self, b2)

  out = pl.pallas_call(
      _out_kernel,
      grid=(grid,),
      in_specs=[
          pl.BlockSpec((_BLK, d_out), lambda i: (i, 0)),
          pl.BlockSpec((2, _BLK, d_hid), lambda i: (0, i, 0)),
          pl.BlockSpec((2, _BLK, 2), lambda i: (0, i, 0)),
          pl.BlockSpec((d_hid, d_out), lambda i: (0, 0)),
      ],
      out_specs=pl.BlockSpec((_BLK, d_out), lambda i: (i, 0)),
      out_shape=jax.ShapeDtypeStruct((n, d_out), jnp.float32),
  )(selfp, ns_parts, deg_parts, W_neigh)

  return out
